# Initial kernel scaffold; baseline (speedup 1.0000x reference)
import jax
import jax.numpy as jnp
from jax.experimental import pallas as pl


def kernel(x, coord, edge_attr, edge_index, pe_w1, pe_b1, pe_w2, pe_b2, pe_p_w1, pe_p_b1, pe_p_w2, pe_p_b2, ni_w, ni_b, mm_w1, mm_b1, mm_w2, mm_b2, geo_w, nm_w1, nm_b1, nm_w2, nm_b2, cm_w1, cm_b1, cm_w2, frequencies):
    raise NotImplementedError("write your pallas kernel here")


# speedup vs baseline: 22.3859x; 22.3859x over previous
import functools

import numpy as np
import jax
import jax.numpy as jnp
from jax import lax
from jax.experimental import pallas as pl
from jax.experimental.pallas import tpu as pltpu
from jax.experimental.pallas import tpu_sc as plsc

N = 10000
E = 160000
D = 128
H = 128
NC = 4
ED = 16
NFB = 32

NP = 10240
NSC = 2
NSUB = 16
NW = NSC * NSUB
EW = E // NW
CH = 128
NFULL = EW // CH
TAIL = EW - NFULL * CH

ROWS_PER_SUB = NP // NSUB


def _silu(v):
    return v * (1.0 / (1.0 + jnp.exp(-v)))


def _build_consts():
    g1 = np.zeros((16, 48), np.float32)
    g2 = np.zeros((16, 48), np.float32)
    s = np.zeros((48, 16), np.float32)
    for i in range(4):
        for k in range(4):
            for j in range(3):
                p = (i * 4 + k) * 3 + j
                g1[3 * i + j, p] = 1.0
                g2[3 * k + j, p] = 1.0
                s[p, i * 4 + k] = 1.0
    r = np.zeros((4, 12), np.float32)
    for i in range(4):
        for j in range(3):
            r[i, 3 * i + j] = 1.0
    return g1, g2, s, r


_G1, _G2, _S, _R = _build_consts()


def _k1_body(x_ref, ws_ref, wt_ref, b_ref, xs_ref, xt_ref):
    x = x_ref[...]
    xs_ref[...] = jnp.dot(x, ws_ref[...], preferred_element_type=jnp.float32) + b_ref[...]
    xt_ref[...] = jnp.dot(x, wt_ref[...], preferred_element_type=jnp.float32)


def _k1(x, ws, wt, nib):
    return pl.pallas_call(
        _k1_body,
        out_shape=(
            jax.ShapeDtypeStruct((N, D), jnp.float32),
            jax.ShapeDtypeStruct((N, D), jnp.float32),
        ),
    )(x, ws, wt, nib)


def _gather_body(row_hbm, col_hbm, xs_hbm, xt_hbm, cp_hbm, ni_out, cd_out,
                 ridx, cidx, r8, c8, a_v, b_v, p_v, q_v, sem):
    wid = lax.axis_index("s") * NSC + lax.axis_index("c")
    base = wid * EW

    def do_chunk(goff, idx_r, idx_c, size):
        pltpu.sync_copy(row_hbm.at[pl.ds(goff, size)], idx_r)
        pltpu.sync_copy(col_hbm.at[pl.ds(goff, size)], idx_c)
        d1 = pltpu.async_copy(xs_hbm.at[idx_r], a_v.at[pl.ds(0, size)], sem)
        d2 = pltpu.async_copy(xt_hbm.at[idx_c], b_v.at[pl.ds(0, size)], sem)
        d3 = pltpu.async_copy(cp_hbm.at[idx_r], p_v.at[pl.ds(0, size)], sem)
        d4 = pltpu.async_copy(cp_hbm.at[idx_c], q_v.at[pl.ds(0, size)], sem)
        d1.wait()
        d2.wait()
        d3.wait()
        d4.wait()

        def body(rr, carry):
            for j in range(8):
                sl = pl.ds(16 * j, 16)
                a_v[rr, sl] = a_v[rr, sl] + b_v[rr, sl]
            p_v[rr, :] = p_v[rr, :] - q_v[rr, :]
            return carry

        lax.fori_loop(0, size, body, 0)
        pltpu.sync_copy(a_v.at[pl.ds(0, size)], ni_out.at[pl.ds(goff, size)])
        pltpu.sync_copy(p_v.at[pl.ds(0, size)], cd_out.at[pl.ds(goff, size)])

    def loop_body(k, carry):
        do_chunk(base + k * CH, ridx, cidx, CH)
        return carry

    lax.fori_loop(0, NFULL, loop_body, 0)
    do_chunk(base + NFULL * CH, r8, c8, TAIL)


def _gather_sc(row, col, xs, xt, coord16):
    mesh = plsc.VectorSubcoreMesh(
        core_axis_name="c", subcore_axis_name="s",
        num_cores=NSC, num_subcores=NSUB)
    fn = functools.partial(
        pl.kernel,
        out_type=(
            jax.ShapeDtypeStruct((E, D), jnp.float32),
            jax.ShapeDtypeStruct((E, 16), jnp.float32),
        ),
        mesh=mesh,
        scratch_types=[
            pltpu.VMEM((CH,), jnp.int32),
            pltpu.VMEM((CH,), jnp.int32),
            pltpu.VMEM((TAIL,), jnp.int32),
            pltpu.VMEM((TAIL,), jnp.int32),
            pltpu.VMEM((CH, D), jnp.float32),
            pltpu.VMEM((CH, D), jnp.float32),
            pltpu.VMEM((CH, 16), jnp.float32),
            pltpu.VMEM((CH, 16), jnp.float32),
            pltpu.SemaphoreType.DMA,
        ],
        compiler_params=pltpu.CompilerParams(use_tc_tiling_on_sc=False),
    )(_gather_body)
    return fn(row, col, xs, xt, coord16)


_P1C = 2000


def _p1_body(cd_ref, g1_ref, g2_ref, s_ref, out_ref):
    cd = cd_ref[...]
    u = jnp.dot(cd, g1_ref[...], preferred_element_type=jnp.float32)
    v = jnp.dot(cd, g2_ref[...], preferred_element_type=jnp.float32)
    rad = jnp.dot(u * v, s_ref[...], preferred_element_type=jnp.float32)
    part = jnp.sum(rad * rad, axis=0, keepdims=True)

    @pl.when(pl.program_id(0) == 0)
    def _():
        out_ref[...] = jnp.zeros_like(out_ref)

    out_ref[...] += part


def _p1(cd16):
    grid = E // _P1C
    return pl.pallas_call(
        _p1_body,
        grid=(grid,),
        in_specs=[
            pl.BlockSpec((_P1C, 16), lambda i: (i, 0)),
            pl.BlockSpec((16, 48), lambda i: (0, 0)),
            pl.BlockSpec((16, 48), lambda i: (0, 0)),
            pl.BlockSpec((48, 16), lambda i: (0, 0)),
        ],
        out_specs=pl.BlockSpec((1, 16), lambda i: (0, 0)),
        out_shape=jax.ShapeDtypeStruct((1, 16), jnp.float32),
    )(cd16, jnp.asarray(_G1), jnp.asarray(_G2), jnp.asarray(_S))


_P2C = 1000


def _p2_body(ni_ref, cd_ref, ea_ref, g1_ref, g2_ref, s_ref,
             w1s_ref, b1_ref, w2_ref, b2_ref,
             scm_ref, ph_ref, w1sc_ref, w1dist_ref, w1dir_ref, pb1_ref,
             wp2_ref, pb2_ref,
             ma_ref, mb_ref, mc_ref, mb1_ref, mw2_ref, mb2_ref,
             cw1_ref, cb1_ref, c2r_ref,
             m_ref, t_ref):
    cd = cd_ref[...]
    u = jnp.dot(cd, g1_ref[...], preferred_element_type=jnp.float32)
    v = jnp.dot(cd, g2_ref[...], preferred_element_type=jnp.float32)
    rad = jnp.dot(u * v, s_ref[...], preferred_element_type=jnp.float32)
    h1 = _silu(jnp.dot(rad, w1s_ref[...], preferred_element_type=jnp.float32) + b1_ref[...])
    cdiff = jnp.dot(h1, w2_ref[...], preferred_element_type=jnp.float32) + b2_ref[...]
    d2 = jnp.sum(cdiff * cdiff, axis=1, keepdims=True)
    dist = jnp.sqrt(d2)
    direction = cdiff * (1.0 / (dist + 1e-8))
    sincos = jnp.sin(jnp.dot(cdiff, scm_ref[...], preferred_element_type=jnp.float32) + ph_ref[...])
    enc1 = (jnp.dot(sincos, w1sc_ref[...], preferred_element_type=jnp.float32)
            + dist * w1dist_ref[...]
            + jnp.dot(direction, w1dir_ref[...], preferred_element_type=jnp.float32)
            + pb1_ref[...])
    pos = jnp.dot(_silu(enc1), wp2_ref[...], preferred_element_type=jnp.float32) + pb2_ref[...]
    ni = _silu(ni_ref[...])
    m1 = _silu(jnp.dot(ni, ma_ref[...], preferred_element_type=jnp.float32)
               + jnp.dot(pos, mb_ref[...], preferred_element_type=jnp.float32)
               + jnp.dot(ea_ref[...], mc_ref[...], preferred_element_type=jnp.float32)
               + mb1_ref[...])
    m = _silu(jnp.dot(m1, mw2_ref[...], preferred_element_type=jnp.float32) + mb2_ref[...])
    m_ref[...] = m
    s1 = _silu(jnp.dot(m, cw1_ref[...], preferred_element_type=jnp.float32) + cb1_ref[...])
    scale16 = jnp.dot(s1, c2r_ref[...], preferred_element_type=jnp.float32)
    lane = lax.broadcasted_iota(jnp.int32, (_P2C, 16), 1)
    ones12 = jnp.where(lane == 12, 1.0, 0.0).astype(jnp.float32)
    t_ref[...] = cd * scale16 + ones12


def _p2(pre_ni, cd16, edge_attr, w1s, b1, w2p, b2p, scm, ph, w1sc, w1dist,
        w1dir, pb1, wp2, pb2, ma, mb, mc, mb1, mw2, mb2, cw1, cb1, c2r):
    grid = E // _P2C
    full = lambda shape: pl.BlockSpec(shape, lambda i: tuple(0 for _ in shape))
    return pl.pallas_call(
        _p2_body,
        grid=(grid,),
        in_specs=[
            pl.BlockSpec((_P2C, D), lambda i: (i, 0)),
            pl.BlockSpec((_P2C, 16), lambda i: (i, 0)),
            pl.BlockSpec((_P2C, ED), lambda i: (i, 0)),
            full((16, 48)), full((16, 48)), full((48, 16)),
            full((16, D)), full((1, D)), full((D, D)), full((1, D)),
            full((D, 192)), full((1, 192)), full((192, 32)), full((1, 32)),
            full((D, 32)), full((1, 32)),
            full((32, 32)), full((1, 32)),
            full((D, D)), full((32, D)), full((ED, D)), full((1, D)),
            full((D, D)), full((1, D)),
            full((D, D)), full((1, D)), full((D, 16)),
        ],
        out_specs=(
            pl.BlockSpec((_P2C, D), lambda i: (i, 0)),
            pl.BlockSpec((_P2C, 16), lambda i: (i, 0)),
        ),
        out_shape=(
            jax.ShapeDtypeStruct((E, D), jnp.float32),
            jax.ShapeDtypeStruct((E, 16), jnp.float32),
        ),
    )(pre_ni, cd16, edge_attr, jnp.asarray(_G1), jnp.asarray(_G2),
      jnp.asarray(_S), w1s, b1, w2p, b2p, scm, ph, w1sc, w1dist,
      w1dir, pb1, wp2, pb2, ma, mb, mc, mb1, mw2, mb2, cw1, cb1, c2r)


def _scatter_body(row_hbm, m_hbm, t_hbm, pm_out, pt_out,
                  idx, idx8, bm, bt, shm, sht):
    cid = lax.axis_index("c")
    sid = lax.axis_index("s")
    wid = sid * NSC + cid
    base = wid * EW
    roff = sid * ROWS_PER_SUB

    zero16 = jnp.zeros((16,), jnp.float32)

    def zb(rr, carry):
        for j in range(8):
            bm[rr, pl.ds(16 * j, 16)] = zero16
        bt[rr, :] = zero16
        return carry

    lax.fori_loop(0, CH, zb, 0)
    for k in range(ROWS_PER_SUB // CH):
        sl = pl.ds(roff + k * CH, CH)
        pltpu.sync_copy(bm, shm.at[sl])
        pltpu.sync_copy(bt, sht.at[sl])
    plsc.subcore_barrier()

    def loop_body(k, carry):
        goff = base + k * CH
        pltpu.sync_copy(row_hbm.at[pl.ds(goff, CH)], idx)
        pltpu.sync_copy(m_hbm.at[pl.ds(goff, CH)], bm)
        pltpu.sync_copy(t_hbm.at[pl.ds(goff, CH)], bt)
        pltpu.sync_copy(bm, shm.at[idx], add=True)
        pltpu.sync_copy(bt, sht.at[idx], add=True)
        return carry

    lax.fori_loop(0, NFULL, loop_body, 0)
    goff = base + NFULL * CH
    pltpu.sync_copy(row_hbm.at[pl.ds(goff, TAIL)], idx8)
    pltpu.sync_copy(m_hbm.at[pl.ds(goff, TAIL)], bm.at[pl.ds(0, TAIL)])
    pltpu.sync_copy(t_hbm.at[pl.ds(goff, TAIL)], bt.at[pl.ds(0, TAIL)])
    pltpu.sync_copy(bm.at[pl.ds(0, TAIL)], shm.at[idx8], add=True)
    pltpu.sync_copy(bt.at[pl.ds(0, TAIL)], sht.at[idx8], add=True)
    plsc.subcore_barrier()

    for k in range(ROWS_PER_SUB // CH):
        sl = pl.ds(roff + k * CH, CH)
        pltpu.sync_copy(shm.at[sl], bm)
        pltpu.sync_copy(sht.at[sl], bt)
        pltpu.sync_copy(bm, pm_out.at[cid, sl])
        pltpu.sync_copy(bt, pt_out.at[cid, sl])


def _scatter_sc(row, m, t16):
    mesh = plsc.VectorSubcoreMesh(
        core_axis_name="c", subcore_axis_name="s",
        num_cores=NSC, num_subcores=NSUB)
    fn = functools.partial(
        pl.kernel,
        out_type=(
            jax.ShapeDtypeStruct((NSC, NP, D), jnp.float32),
            jax.ShapeDtypeStruct((NSC, NP, 16), jnp.float32),
        ),
        mesh=mesh,
        scratch_types=[
            pltpu.VMEM((CH,), jnp.int32),
            pltpu.VMEM((TAIL,), jnp.int32),
            pltpu.VMEM((CH, D), jnp.float32),
            pltpu.VMEM((CH, 16), jnp.float32),
            pltpu.VMEM_SHARED((NP, D), jnp.float32),
            pltpu.VMEM_SHARED((NP, 16), jnp.float32),
        ],
        compiler_params=pltpu.CompilerParams(use_tc_tiling_on_sc=False),
    )(_scatter_body)
    return fn(row, m, t16)


def _k5_body(x_ref, c16_ref, pm_ref, pt_ref, geo_ref,
             nx_ref, na_ref, nb1_ref, nw2_ref, nb2_ref, h_ref, co_ref):
    aggm = (pm_ref[0] + pm_ref[1])[:N]
    agg = jnp.dot(aggm, geo_ref[...], preferred_element_type=jnp.float32)
    x = x_ref[...]
    h1 = _silu(jnp.dot(x, nx_ref[...], preferred_element_type=jnp.float32)
               + jnp.dot(agg, na_ref[...], preferred_element_type=jnp.float32)
               + nb1_ref[...])
    h_ref[...] = x + jnp.dot(h1, nw2_ref[...], preferred_element_type=jnp.float32) + nb2_ref[...]
    qt = (pt_ref[0] + pt_ref[1])[:N]
    cnt = qt[:, 12:13]
    lane = lax.broadcasted_iota(jnp.int32, (N, 16), 1)
    tr = jnp.where(lane < 12, qt, 0.0)
    co_ref[...] = c16_ref[...] + tr * (1.0 / jnp.maximum(cnt, 1.0))


def _k5(x, coord16, pm, pt, geo, nx, na, nb1, nw2, nb2):
    return pl.pallas_call(
        _k5_body,
        out_shape=(
            jax.ShapeDtypeStruct((N, D), jnp.float32),
            jax.ShapeDtypeStruct((N, 16), jnp.float32),
        ),
    )(x, coord16, pm, pt, geo, nx, na, nb1, nw2, nb2)


def kernel(x, coord, edge_attr, edge_index, pe_w1, pe_b1, pe_w2, pe_b2,
           pe_p_w1, pe_p_b1, pe_p_w2, pe_p_b2, ni_w, ni_b, mm_w1, mm_b1,
           mm_w2, mm_b2, geo_w, nm_w1, nm_b1, nm_w2, nm_b2, cm_w1, cm_b1,
           cm_w2, frequencies):
    f32 = jnp.float32
    row = edge_index[0]
    col = edge_index[1]
    coord16 = jnp.pad(coord.reshape(N, 12), ((0, 0), (0, 4))).astype(f32)

    ni_wT = ni_w.T
    ws, wt = ni_wT[:D], ni_wT[D:]
    nib = ni_b.reshape(1, D)

    pe_w1T = pe_w1.T
    b1 = pe_b1.reshape(1, D)
    w2p = jnp.pad(pe_w2.T, ((0, 0), (0, D - 3)))
    b2p = jnp.pad(pe_b2.reshape(1, 3), ((0, 0), (0, D - 3)))

    m3 = jnp.kron(jnp.eye(3, dtype=f32), frequencies.reshape(1, NFB))
    scm = jnp.pad(jnp.concatenate([m3, m3], axis=1), ((0, D - 3), (0, 0)))
    ph = jnp.concatenate([jnp.zeros((1, 96), f32),
                          jnp.full((1, 96), np.float32(np.pi / 2))], axis=1)

    w1T = pe_p_w1.T
    w1sc = w1T[:192]
    w1dist = w1T[192:193]
    w1dir = jnp.pad(w1T[193:196], ((0, D - 3), (0, 0)))
    pb1 = pe_p_b1.reshape(1, 32)
    wp2 = pe_p_w2.T
    pb2 = pe_p_b2.reshape(1, 32)

    mm_w1T = mm_w1.T
    ma, mb, mc = mm_w1T[:D], mm_w1T[D:D + 32], mm_w1T[D + 32:]
    mb1 = mm_b1.reshape(1, D)
    mw2 = mm_w2.T
    mb2 = mm_b2.reshape(1, D)

    cw1 = cm_w1.T
    cb1 = cm_b1.reshape(1, D)
    c2r = jnp.pad(cm_w2.T @ jnp.asarray(_R), ((0, 0), (0, 4)))

    geo = geo_w.T
    nm_w1T = nm_w1.T
    nx, na = nm_w1T[:D], nm_w1T[D:]
    nb1 = nm_b1.reshape(1, D)
    nw2 = nm_w2.T
    nb2 = nm_b2.reshape(1, D)

    xs, xt = _k1(x, ws, wt, nib)
    pre_ni, cd16 = _gather_sc(row, col, xs, xt, coord16)
    sumsq = _p1(cd16)
    nrm = jnp.sqrt(sumsq.reshape(16))
    w1s = pe_w1T * (1.0 / jnp.maximum(nrm, 1e-12))[:, None]
    m, t16 = _p2(pre_ni, cd16, edge_attr, w1s, b1, w2p, b2p, scm, ph,
                 w1sc, w1dist, w1dir, pb1, wp2, pb2,
                 ma, mb, mc, mb1, mw2, mb2, cw1, cb1, c2r)
    pm, pt = _scatter_sc(row, m, t16)
    h_out, co16 = _k5(x, coord16, pm, pt, geo, nx, na, nb1, nw2, nb2)
    coord_out = co16[:, :12].reshape(N, NC, 3)
    return (h_out, coord_out)


# speedup vs baseline: 29.8739x; 1.3345x over previous
import functools

import numpy as np
import jax
import jax.numpy as jnp
from jax import lax
from jax.experimental import pallas as pl
from jax.experimental.pallas import tpu as pltpu
from jax.experimental.pallas import tpu_sc as plsc

N = 10000
E = 160000
D = 128
H = 128
NC = 4
ED = 16
NFB = 32

NP = 10240
NSC = 2
NSUB = 16
NW = NSC * NSUB
EW = E // NW
CH = 128
NFULL = EW // CH
TAIL = EW - NFULL * CH

ROWS_PER_SUB = NP // NSUB


def _silu(v):
    return v * (1.0 / (1.0 + jnp.exp(-v)))


_TWO_PI = np.float32(2.0 * np.pi)
_INV_TWO_PI = np.float32(1.0 / (2.0 * np.pi))
_PI = np.float32(np.pi)
_HALF_PI = np.float32(np.pi / 2.0)


def _fast_sin(x):
    y = x * _INV_TWO_PI
    t = (y - jnp.round(y)) * _TWO_PI
    at = jnp.abs(t)
    f = jnp.where(at > _HALF_PI, _PI - at, at)
    f2 = f * f
    p = f * (1.0 + f2 * (np.float32(-1.0 / 6.0)
                         + f2 * (np.float32(1.0 / 120.0)
                                 + f2 * (np.float32(-1.0 / 5040.0)
                                         + f2 * np.float32(1.0 / 362880.0)))))
    return jnp.where(t < 0.0, -p, p)


def _build_consts():
    g1 = np.zeros((16, 48), np.float32)
    g2 = np.zeros((16, 48), np.float32)
    s = np.zeros((48, 16), np.float32)
    for i in range(4):
        for k in range(4):
            for j in range(3):
                p = (i * 4 + k) * 3 + j
                g1[3 * i + j, p] = 1.0
                g2[3 * k + j, p] = 1.0
                s[p, i * 4 + k] = 1.0
    r = np.zeros((4, 12), np.float32)
    for i in range(4):
        for j in range(3):
            r[i, 3 * i + j] = 1.0
    return g1, g2, s, r


_G1, _G2, _S, _R = _build_consts()


def _k1_body(x_ref, ws_ref, wt_ref, b_ref, xs_ref, xt_ref):
    x = x_ref[...]
    xs_ref[...] = jnp.dot(x, ws_ref[...], preferred_element_type=jnp.float32) + b_ref[...]
    xt_ref[...] = jnp.dot(x, wt_ref[...], preferred_element_type=jnp.float32)


def _k1(x, ws, wt, nib):
    return pl.pallas_call(
        _k1_body,
        out_shape=(
            jax.ShapeDtypeStruct((N, D), jnp.float32),
            jax.ShapeDtypeStruct((N, D), jnp.float32),
        ),
    )(x, ws, wt, nib)


def _gather_body(row_hbm, col_hbm, xs_hbm, xt_hbm, cp_hbm, ni_out, cd_out,
                 ridx, cidx, r8, c8, a_v, b_v, p_v, q_v, sem):
    wid = lax.axis_index("s") * NSC + lax.axis_index("c")
    base = wid * EW

    def do_chunk(goff, idx_r, idx_c, size):
        pltpu.sync_copy(row_hbm.at[pl.ds(goff, size)], idx_r)
        pltpu.sync_copy(col_hbm.at[pl.ds(goff, size)], idx_c)
        d1 = pltpu.async_copy(xs_hbm.at[idx_r], a_v.at[pl.ds(0, size)], sem)
        d2 = pltpu.async_copy(xt_hbm.at[idx_c], b_v.at[pl.ds(0, size)], sem)
        d3 = pltpu.async_copy(cp_hbm.at[idx_r], p_v.at[pl.ds(0, size)], sem)
        d4 = pltpu.async_copy(cp_hbm.at[idx_c], q_v.at[pl.ds(0, size)], sem)
        d1.wait()
        d2.wait()
        d3.wait()
        d4.wait()

        def body(rr, carry):
            for j in range(8):
                sl = pl.ds(16 * j, 16)
                a_v[rr, sl] = a_v[rr, sl] + b_v[rr, sl]
            p_v[rr, :] = p_v[rr, :] - q_v[rr, :]
            return carry

        lax.fori_loop(0, size, body, 0)
        pltpu.sync_copy(a_v.at[pl.ds(0, size)], ni_out.at[pl.ds(goff, size)])
        pltpu.sync_copy(p_v.at[pl.ds(0, size)], cd_out.at[pl.ds(goff, size)])

    def loop_body(k, carry):
        do_chunk(base + k * CH, ridx, cidx, CH)
        return carry

    lax.fori_loop(0, NFULL, loop_body, 0)
    do_chunk(base + NFULL * CH, r8, c8, TAIL)


def _gather_sc(row, col, xs, xt, coord16):
    mesh = plsc.VectorSubcoreMesh(
        core_axis_name="c", subcore_axis_name="s",
        num_cores=NSC, num_subcores=NSUB)
    fn = functools.partial(
        pl.kernel,
        out_type=(
            jax.ShapeDtypeStruct((E, D), jnp.float32),
            jax.ShapeDtypeStruct((E, 16), jnp.float32),
        ),
        mesh=mesh,
        scratch_types=[
            pltpu.VMEM((CH,), jnp.int32),
            pltpu.VMEM((CH,), jnp.int32),
            pltpu.VMEM((TAIL,), jnp.int32),
            pltpu.VMEM((TAIL,), jnp.int32),
            pltpu.VMEM((CH, D), jnp.float32),
            pltpu.VMEM((CH, D), jnp.float32),
            pltpu.VMEM((CH, 16), jnp.float32),
            pltpu.VMEM((CH, 16), jnp.float32),
            pltpu.SemaphoreType.DMA,
        ],
        compiler_params=pltpu.CompilerParams(use_tc_tiling_on_sc=False),
    )(_gather_body)
    return fn(row, col, xs, xt, coord16)


_P1C = 2000


def _p1_body(cd_ref, g1_ref, g2_ref, s_ref, out_ref):
    cd = cd_ref[...]
    u = jnp.dot(cd, g1_ref[...], preferred_element_type=jnp.float32)
    v = jnp.dot(cd, g2_ref[...], preferred_element_type=jnp.float32)
    rad = jnp.dot(u * v, s_ref[...], preferred_element_type=jnp.float32)
    part = jnp.sum(rad * rad, axis=0, keepdims=True)

    @pl.when(pl.program_id(0) == 0)
    def _():
        out_ref[...] = jnp.zeros_like(out_ref)

    out_ref[...] += part


def _p1(cd16):
    grid = E // _P1C
    return pl.pallas_call(
        _p1_body,
        grid=(grid,),
        in_specs=[
            pl.BlockSpec((_P1C, 16), lambda i: (i, 0)),
            pl.BlockSpec((16, 48), lambda i: (0, 0)),
            pl.BlockSpec((16, 48), lambda i: (0, 0)),
            pl.BlockSpec((48, 16), lambda i: (0, 0)),
        ],
        out_specs=pl.BlockSpec((1, 16), lambda i: (0, 0)),
        out_shape=jax.ShapeDtypeStruct((1, 16), jnp.float32),
    )(cd16, jnp.asarray(_G1), jnp.asarray(_G2), jnp.asarray(_S))


_P2C = 1000


def _p2_body(ni_ref, cd_ref, ea_ref, g1_ref, g2_ref, s_ref,
             w1s_ref, b1_ref, w2_ref, b2_ref,
             scm_ref, ph_ref, w1sc_ref, w1dist_ref, w1dir_ref, pb1_ref,
             wp2_ref, pb2_ref,
             ma_ref, mb_ref, mc_ref, mb1_ref, mw2_ref, mb2_ref,
             cw1_ref, cb1_ref, c2r_ref,
             m_ref, t_ref):
    cd = cd_ref[...]
    u = jnp.dot(cd, g1_ref[...], preferred_element_type=jnp.float32)
    v = jnp.dot(cd, g2_ref[...], preferred_element_type=jnp.float32)
    rad = jnp.dot(u * v, s_ref[...], preferred_element_type=jnp.float32)
    h1 = _silu(jnp.dot(rad, w1s_ref[...], preferred_element_type=jnp.float32) + b1_ref[...])
    cdiff = jnp.dot(h1, w2_ref[...], preferred_element_type=jnp.float32) + b2_ref[...]
    d2 = jnp.sum(cdiff * cdiff, axis=1, keepdims=True)
    dist = jnp.sqrt(d2)
    direction = cdiff * (1.0 / (dist + 1e-8))
    sincos = _fast_sin(jnp.dot(cdiff, scm_ref[...], preferred_element_type=jnp.float32) + ph_ref[...])
    enc1 = (jnp.dot(sincos, w1sc_ref[...], preferred_element_type=jnp.float32)
            + dist * w1dist_ref[...]
            + jnp.dot(direction, w1dir_ref[...], preferred_element_type=jnp.float32)
            + pb1_ref[...])
    pos = jnp.dot(_silu(enc1), wp2_ref[...], preferred_element_type=jnp.float32) + pb2_ref[...]
    ni = _silu(ni_ref[...])
    m1 = _silu(jnp.dot(ni, ma_ref[...], preferred_element_type=jnp.float32)
               + jnp.dot(pos, mb_ref[...], preferred_element_type=jnp.float32)
               + jnp.dot(ea_ref[...], mc_ref[...], preferred_element_type=jnp.float32)
               + mb1_ref[...])
    m = _silu(jnp.dot(m1, mw2_ref[...], preferred_element_type=jnp.float32) + mb2_ref[...])
    m_ref[...] = m
    s1 = _silu(jnp.dot(m, cw1_ref[...], preferred_element_type=jnp.float32) + cb1_ref[...])
    scale16 = jnp.dot(s1, c2r_ref[...], preferred_element_type=jnp.float32)
    lane = lax.broadcasted_iota(jnp.int32, (_P2C, 16), 1)
    ones12 = jnp.where(lane == 12, 1.0, 0.0).astype(jnp.float32)
    t_ref[...] = cd * scale16 + ones12


def _p2(pre_ni, cd16, edge_attr, w1s, b1, w2p, b2p, scm, ph, w1sc, w1dist,
        w1dir, pb1, wp2, pb2, ma, mb, mc, mb1, mw2, mb2, cw1, cb1, c2r):
    grid = E // _P2C
    full = lambda shape: pl.BlockSpec(shape, lambda i: tuple(0 for _ in shape))
    return pl.pallas_call(
        _p2_body,
        grid=(grid,),
        in_specs=[
            pl.BlockSpec((_P2C, D), lambda i: (i, 0)),
            pl.BlockSpec((_P2C, 16), lambda i: (i, 0)),
            pl.BlockSpec((_P2C, ED), lambda i: (i, 0)),
            full((16, 48)), full((16, 48)), full((48, 16)),
            full((16, D)), full((1, D)), full((D, D)), full((1, D)),
            full((D, 192)), full((1, 192)), full((192, 32)), full((1, 32)),
            full((D, 32)), full((1, 32)),
            full((32, 32)), full((1, 32)),
            full((D, D)), full((32, D)), full((ED, D)), full((1, D)),
            full((D, D)), full((1, D)),
            full((D, D)), full((1, D)), full((D, 16)),
        ],
        out_specs=(
            pl.BlockSpec((_P2C, D), lambda i: (i, 0)),
            pl.BlockSpec((_P2C, 16), lambda i: (i, 0)),
        ),
        out_shape=(
            jax.ShapeDtypeStruct((E, D), jnp.float32),
            jax.ShapeDtypeStruct((E, 16), jnp.float32),
        ),
    )(pre_ni, cd16, edge_attr, jnp.asarray(_G1), jnp.asarray(_G2),
      jnp.asarray(_S), w1s, b1, w2p, b2p, scm, ph, w1sc, w1dist,
      w1dir, pb1, wp2, pb2, ma, mb, mc, mb1, mw2, mb2, cw1, cb1, c2r)


def _scatter_body(row_hbm, m_hbm, t_hbm, pm_out, pt_out,
                  idx, idx8, bm, bt, shm, sht):
    cid = lax.axis_index("c")
    sid = lax.axis_index("s")
    wid = sid * NSC + cid
    base = wid * EW
    roff = sid * ROWS_PER_SUB

    zero16 = jnp.zeros((16,), jnp.float32)

    def zb(rr, carry):
        for j in range(8):
            bm[rr, pl.ds(16 * j, 16)] = zero16
        bt[rr, :] = zero16
        return carry

    lax.fori_loop(0, CH, zb, 0)
    for k in range(ROWS_PER_SUB // CH):
        sl = pl.ds(roff + k * CH, CH)
        pltpu.sync_copy(bm, shm.at[sl])
        pltpu.sync_copy(bt, sht.at[sl])
    plsc.subcore_barrier()

    def loop_body(k, carry):
        goff = base + k * CH
        pltpu.sync_copy(row_hbm.at[pl.ds(goff, CH)], idx)
        pltpu.sync_copy(m_hbm.at[pl.ds(goff, CH)], bm)
        pltpu.sync_copy(t_hbm.at[pl.ds(goff, CH)], bt)
        pltpu.sync_copy(bm, shm.at[idx], add=True)
        pltpu.sync_copy(bt, sht.at[idx], add=True)
        return carry

    lax.fori_loop(0, NFULL, loop_body, 0)
    goff = base + NFULL * CH
    pltpu.sync_copy(row_hbm.at[pl.ds(goff, TAIL)], idx8)
    pltpu.sync_copy(m_hbm.at[pl.ds(goff, TAIL)], bm.at[pl.ds(0, TAIL)])
    pltpu.sync_copy(t_hbm.at[pl.ds(goff, TAIL)], bt.at[pl.ds(0, TAIL)])
    pltpu.sync_copy(bm.at[pl.ds(0, TAIL)], shm.at[idx8], add=True)
    pltpu.sync_copy(bt.at[pl.ds(0, TAIL)], sht.at[idx8], add=True)
    plsc.subcore_barrier()

    for k in range(ROWS_PER_SUB // CH):
        sl = pl.ds(roff + k * CH, CH)
        pltpu.sync_copy(shm.at[sl], bm)
        pltpu.sync_copy(sht.at[sl], bt)
        pltpu.sync_copy(bm, pm_out.at[cid, sl])
        pltpu.sync_copy(bt, pt_out.at[cid, sl])


def _scatter_sc(row, m, t16):
    mesh = plsc.VectorSubcoreMesh(
        core_axis_name="c", subcore_axis_name="s",
        num_cores=NSC, num_subcores=NSUB)
    fn = functools.partial(
        pl.kernel,
        out_type=(
            jax.ShapeDtypeStruct((NSC, NP, D), jnp.float32),
            jax.ShapeDtypeStruct((NSC, NP, 16), jnp.float32),
        ),
        mesh=mesh,
        scratch_types=[
            pltpu.VMEM((CH,), jnp.int32),
            pltpu.VMEM((TAIL,), jnp.int32),
            pltpu.VMEM((CH, D), jnp.float32),
            pltpu.VMEM((CH, 16), jnp.float32),
            pltpu.VMEM_SHARED((NP, D), jnp.float32),
            pltpu.VMEM_SHARED((NP, 16), jnp.float32),
        ],
        compiler_params=pltpu.CompilerParams(use_tc_tiling_on_sc=False),
    )(_scatter_body)
    return fn(row, m, t16)


def _k5_body(x_ref, c16_ref, pm_ref, pt_ref, geo_ref,
             nx_ref, na_ref, nb1_ref, nw2_ref, nb2_ref, h_ref, co_ref):
    aggm = (pm_ref[0] + pm_ref[1])[:N]
    agg = jnp.dot(aggm, geo_ref[...], preferred_element_type=jnp.float32)
    x = x_ref[...]
    h1 = _silu(jnp.dot(x, nx_ref[...], preferred_element_type=jnp.float32)
               + jnp.dot(agg, na_ref[...], preferred_element_type=jnp.float32)
               + nb1_ref[...])
    h_ref[...] = x + jnp.dot(h1, nw2_ref[...], preferred_element_type=jnp.float32) + nb2_ref[...]
    qt = (pt_ref[0] + pt_ref[1])[:N]
    cnt = qt[:, 12:13]
    lane = lax.broadcasted_iota(jnp.int32, (N, 16), 1)
    tr = jnp.where(lane < 12, qt, 0.0)
    co_ref[...] = c16_ref[...] + tr * (1.0 / jnp.maximum(cnt, 1.0))


def _k5(x, coord16, pm, pt, geo, nx, na, nb1, nw2, nb2):
    return pl.pallas_call(
        _k5_body,
        out_shape=(
            jax.ShapeDtypeStruct((N, D), jnp.float32),
            jax.ShapeDtypeStruct((N, 16), jnp.float32),
        ),
    )(x, coord16, pm, pt, geo, nx, na, nb1, nw2, nb2)


def kernel(x, coord, edge_attr, edge_index, pe_w1, pe_b1, pe_w2, pe_b2,
           pe_p_w1, pe_p_b1, pe_p_w2, pe_p_b2, ni_w, ni_b, mm_w1, mm_b1,
           mm_w2, mm_b2, geo_w, nm_w1, nm_b1, nm_w2, nm_b2, cm_w1, cm_b1,
           cm_w2, frequencies):
    f32 = jnp.float32
    row = edge_index[0]
    col = edge_index[1]
    coord16 = jnp.pad(coord.reshape(N, 12), ((0, 0), (0, 4))).astype(f32)

    ni_wT = ni_w.T
    ws, wt = ni_wT[:D], ni_wT[D:]
    nib = ni_b.reshape(1, D)

    pe_w1T = pe_w1.T
    b1 = pe_b1.reshape(1, D)
    w2p = jnp.pad(pe_w2.T, ((0, 0), (0, D - 3)))
    b2p = jnp.pad(pe_b2.reshape(1, 3), ((0, 0), (0, D - 3)))

    m3 = jnp.kron(jnp.eye(3, dtype=f32), frequencies.reshape(1, NFB))
    scm = jnp.pad(jnp.concatenate([m3, m3], axis=1), ((0, D - 3), (0, 0)))
    ph = jnp.concatenate([jnp.zeros((1, 96), f32),
                          jnp.full((1, 96), np.float32(np.pi / 2))], axis=1)

    w1T = pe_p_w1.T
    w1sc = w1T[:192]
    w1dist = w1T[192:193]
    w1dir = jnp.pad(w1T[193:196], ((0, D - 3), (0, 0)))
    pb1 = pe_p_b1.reshape(1, 32)
    wp2 = pe_p_w2.T
    pb2 = pe_p_b2.reshape(1, 32)

    mm_w1T = mm_w1.T
    ma, mb, mc = mm_w1T[:D], mm_w1T[D:D + 32], mm_w1T[D + 32:]
    mb1 = mm_b1.reshape(1, D)
    mw2 = mm_w2.T
    mb2 = mm_b2.reshape(1, D)

    cw1 = cm_w1.T
    cb1 = cm_b1.reshape(1, D)
    c2r = jnp.pad(cm_w2.T @ jnp.asarray(_R), ((0, 0), (0, 4)))

    geo = geo_w.T
    nm_w1T = nm_w1.T
    nx, na = nm_w1T[:D], nm_w1T[D:]
    nb1 = nm_b1.reshape(1, D)
    nw2 = nm_w2.T
    nb2 = nm_b2.reshape(1, D)

    xs, xt = _k1(x, ws, wt, nib)
    pre_ni, cd16 = _gather_sc(row, col, xs, xt, coord16)
    sumsq = _p1(cd16)
    nrm = jnp.sqrt(sumsq.reshape(16))
    w1s = pe_w1T * (1.0 / jnp.maximum(nrm, 1e-12))[:, None]
    m, t16 = _p2(pre_ni, cd16, edge_attr, w1s, b1, w2p, b2p, scm, ph,
                 w1sc, w1dist, w1dir, pb1, wp2, pb2,
                 ma, mb, mc, mb1, mw2, mb2, cw1, cb1, c2r)
    pm, pt = _scatter_sc(row, m, t16)
    h_out, co16 = _k5(x, coord16, pm, pt, geo, nx, na, nb1, nw2, nb2)
    coord_out = co16[:, :12].reshape(N, NC, 3)
    return (h_out, coord_out)


# speedup vs baseline: 32.4869x; 1.0875x over previous
import functools

import numpy as np
import jax
import jax.numpy as jnp
from jax import lax
from jax.experimental import pallas as pl
from jax.experimental.pallas import tpu as pltpu
from jax.experimental.pallas import tpu_sc as plsc

N = 10000
E = 160000
D = 128
H = 128
NC = 4
ED = 16
NFB = 32

NP = 10240
NSC = 2
NSUB = 16
NW = NSC * NSUB
EW = E // NW
CH = 128
NFULL = EW // CH
TAIL = EW - NFULL * CH

ROWS_PER_SUB = NP // NSUB


def _silu(v):
    return v * (1.0 / (1.0 + jnp.exp(-v)))


_TWO_PI = np.float32(2.0 * np.pi)
_INV_TWO_PI = np.float32(1.0 / (2.0 * np.pi))
_PI = np.float32(np.pi)
_HALF_PI = np.float32(np.pi / 2.0)


def _fast_sin(x):
    y = x * _INV_TWO_PI
    t = (y - jnp.round(y)) * _TWO_PI
    at = jnp.abs(t)
    f = jnp.where(at > _HALF_PI, _PI - at, at)
    f2 = f * f
    p = f * (1.0 + f2 * (np.float32(-1.0 / 6.0)
                         + f2 * (np.float32(1.0 / 120.0)
                                 + f2 * (np.float32(-1.0 / 5040.0)
                                         + f2 * np.float32(1.0 / 362880.0)))))
    return jnp.where(t < 0.0, -p, p)


def _build_consts():
    g1 = np.zeros((16, 48), np.float32)
    g2 = np.zeros((16, 48), np.float32)
    s = np.zeros((48, 16), np.float32)
    for i in range(4):
        for k in range(4):
            for j in range(3):
                p = (i * 4 + k) * 3 + j
                g1[3 * i + j, p] = 1.0
                g2[3 * k + j, p] = 1.0
                s[p, i * 4 + k] = 1.0
    r = np.zeros((4, 12), np.float32)
    for i in range(4):
        for j in range(3):
            r[i, 3 * i + j] = 1.0
    return g1, g2, s, r


_G1, _G2, _S, _R = _build_consts()


def _k1_body(x_ref, ws_ref, wt_ref, b_ref, xs_ref, xt_ref):
    x = x_ref[...]
    xs_ref[...] = jnp.dot(x, ws_ref[...], preferred_element_type=jnp.float32) + b_ref[...]
    xt_ref[...] = jnp.dot(x, wt_ref[...], preferred_element_type=jnp.float32)


def _k1(x, ws, wt, nib):
    return pl.pallas_call(
        _k1_body,
        out_shape=(
            jax.ShapeDtypeStruct((N, D), jnp.float32),
            jax.ShapeDtypeStruct((N, D), jnp.float32),
        ),
    )(x, ws, wt, nib)


def _gather_body(row_hbm, col_hbm, xs_hbm, xt_hbm, cp_hbm, ni_out, cd_out,
                 ridx0, cidx0, ridx1, cidx1, r8, c8,
                 a0, b0, p0, q0, a1, b1, p1, q1, sg0, sg1, so0, so1):
    wid = lax.axis_index("s") * NSC + lax.axis_index("c")
    base = wid * EW

    set0 = (ridx0, cidx0, a0, b0, p0, q0, sg0, so0)
    set1 = (ridx1, cidx1, a1, b1, p1, q1, sg1, so1)

    def load_idx(goff, st):
        ridx, cidx = st[0], st[1]
        pltpu.sync_copy(row_hbm.at[pl.ds(goff, CH)], ridx)
        pltpu.sync_copy(col_hbm.at[pl.ds(goff, CH)], cidx)

    def issue(st):
        ridx, cidx, a, b, p, q, sg, _ = st
        pltpu.async_copy(xs_hbm.at[ridx], a, sg)
        pltpu.async_copy(xt_hbm.at[cidx], b, sg)
        pltpu.async_copy(cp_hbm.at[ridx], p, sg)
        pltpu.async_copy(cp_hbm.at[cidx], q, sg)

    def wait_g(st):
        _, _, a, b, p, q, sg, _ = st
        pltpu.make_async_copy(xs_hbm.at[pl.ds(0, CH)], a, sg).wait()
        pltpu.make_async_copy(xs_hbm.at[pl.ds(0, CH)], b, sg).wait()
        pltpu.make_async_copy(cp_hbm.at[pl.ds(0, CH)], p, sg).wait()
        pltpu.make_async_copy(cp_hbm.at[pl.ds(0, CH)], q, sg).wait()

    def compute(st, size):
        _, _, a, b, p, q, _, _ = st

        def body(rr, carry):
            for j in range(8):
                sl = pl.ds(16 * j, 16)
                a[rr, sl] = a[rr, sl] + b[rr, sl]
            p[rr, :] = p[rr, :] - q[rr, :]
            return carry

        lax.fori_loop(0, size, body, 0)

    def out_async(goff, st):
        _, _, a, _, p, _, _, so = st
        pltpu.async_copy(a, ni_out.at[pl.ds(goff, CH)], so)
        pltpu.async_copy(p, cd_out.at[pl.ds(goff, CH)], so)

    def wait_o(st):
        _, _, a, _, p, _, _, so = st
        pltpu.make_async_copy(a, ni_out.at[pl.ds(0, CH)], so).wait()
        pltpu.make_async_copy(p, cd_out.at[pl.ds(0, CH)], so).wait()

    load_idx(base, set0)
    issue(set0)

    def loop_body(h, carry):
        c0 = base + (2 * h) * CH
        c1 = base + (2 * h + 1) * CH
        wait_g(set0)
        load_idx(c1, set1)
        issue(set1)
        compute(set0, CH)
        out_async(c0, set0)
        wait_g(set1)
        load_idx(c0 + 2 * CH, set0)
        wait_o(set0)
        issue(set0)
        compute(set1, CH)
        out_async(c1, set1)
        wait_o(set1)
        return carry

    lax.fori_loop(0, (NFULL - 1) // 2, loop_body, 0)

    gl = base + (NFULL - 1) * CH
    wait_g(set0)
    compute(set0, CH)
    out_async(gl, set0)
    wait_o(set0)

    gt = base + NFULL * CH
    pltpu.sync_copy(row_hbm.at[pl.ds(gt, TAIL)], r8)
    pltpu.sync_copy(col_hbm.at[pl.ds(gt, TAIL)], c8)
    pltpu.async_copy(xs_hbm.at[r8], a0.at[pl.ds(0, TAIL)], sg0).wait()
    pltpu.async_copy(xt_hbm.at[c8], b0.at[pl.ds(0, TAIL)], sg0).wait()
    pltpu.async_copy(cp_hbm.at[r8], p0.at[pl.ds(0, TAIL)], sg0).wait()
    pltpu.async_copy(cp_hbm.at[c8], q0.at[pl.ds(0, TAIL)], sg0).wait()
    compute(set0, TAIL)
    pltpu.sync_copy(a0.at[pl.ds(0, TAIL)], ni_out.at[pl.ds(gt, TAIL)])
    pltpu.sync_copy(p0.at[pl.ds(0, TAIL)], cd_out.at[pl.ds(gt, TAIL)])


def _gather_sc(row, col, xs, xt, coord16):
    mesh = plsc.VectorSubcoreMesh(
        core_axis_name="c", subcore_axis_name="s",
        num_cores=NSC, num_subcores=NSUB)
    fn = functools.partial(
        pl.kernel,
        out_type=(
            jax.ShapeDtypeStruct((E, D), jnp.float32),
            jax.ShapeDtypeStruct((E, 16), jnp.float32),
        ),
        mesh=mesh,
        scratch_types=[
            pltpu.VMEM((CH,), jnp.int32),
            pltpu.VMEM((CH,), jnp.int32),
            pltpu.VMEM((CH,), jnp.int32),
            pltpu.VMEM((CH,), jnp.int32),
            pltpu.VMEM((TAIL,), jnp.int32),
            pltpu.VMEM((TAIL,), jnp.int32),
            pltpu.VMEM((CH, D), jnp.float32),
            pltpu.VMEM((CH, D), jnp.float32),
            pltpu.VMEM((CH, 16), jnp.float32),
            pltpu.VMEM((CH, 16), jnp.float32),
            pltpu.VMEM((CH, D), jnp.float32),
            pltpu.VMEM((CH, D), jnp.float32),
            pltpu.VMEM((CH, 16), jnp.float32),
            pltpu.VMEM((CH, 16), jnp.float32),
            pltpu.SemaphoreType.DMA,
            pltpu.SemaphoreType.DMA,
            pltpu.SemaphoreType.DMA,
            pltpu.SemaphoreType.DMA,
        ],
        compiler_params=pltpu.CompilerParams(use_tc_tiling_on_sc=False),
    )(_gather_body)
    return fn(row, col, xs, xt, coord16)


_P1C = 2000


def _p1_body(cd_ref, g1_ref, g2_ref, s_ref, out_ref):
    cd = cd_ref[...]
    u = jnp.dot(cd, g1_ref[...], preferred_element_type=jnp.float32)
    v = jnp.dot(cd, g2_ref[...], preferred_element_type=jnp.float32)
    rad = jnp.dot(u * v, s_ref[...], preferred_element_type=jnp.float32)
    part = jnp.sum(rad * rad, axis=0, keepdims=True)

    @pl.when(pl.program_id(0) == 0)
    def _():
        out_ref[...] = jnp.zeros_like(out_ref)

    out_ref[...] += part


def _p1(cd16):
    grid = E // _P1C
    return pl.pallas_call(
        _p1_body,
        grid=(grid,),
        in_specs=[
            pl.BlockSpec((_P1C, 16), lambda i: (i, 0)),
            pl.BlockSpec((16, 48), lambda i: (0, 0)),
            pl.BlockSpec((16, 48), lambda i: (0, 0)),
            pl.BlockSpec((48, 16), lambda i: (0, 0)),
        ],
        out_specs=pl.BlockSpec((1, 16), lambda i: (0, 0)),
        out_shape=jax.ShapeDtypeStruct((1, 16), jnp.float32),
    )(cd16, jnp.asarray(_G1), jnp.asarray(_G2), jnp.asarray(_S))


_P2C = 1000


def _p2_body(ni_ref, cd_ref, ea_ref, g1_ref, g2_ref, s_ref,
             w1s_ref, b1_ref, w2_ref, b2_ref,
             scm_ref, ph_ref, w1sc_ref, w1dist_ref, w1dir_ref, pb1_ref,
             wp2_ref, pb2_ref,
             ma_ref, mb_ref, mc_ref, mb1_ref, mw2_ref, mb2_ref,
             cw1_ref, cb1_ref, c2r_ref,
             m_ref, t_ref):
    cd = cd_ref[...]
    u = jnp.dot(cd, g1_ref[...], preferred_element_type=jnp.float32)
    v = jnp.dot(cd, g2_ref[...], preferred_element_type=jnp.float32)
    rad = jnp.dot(u * v, s_ref[...], preferred_element_type=jnp.float32)
    h1 = _silu(jnp.dot(rad, w1s_ref[...], preferred_element_type=jnp.float32) + b1_ref[...])
    cdiff = jnp.dot(h1, w2_ref[...], preferred_element_type=jnp.float32) + b2_ref[...]
    d2 = jnp.sum(cdiff * cdiff, axis=1, keepdims=True)
    dist = jnp.sqrt(d2)
    direction = cdiff * (1.0 / (dist + 1e-8))
    sincos = _fast_sin(jnp.dot(cdiff, scm_ref[...], preferred_element_type=jnp.float32) + ph_ref[...])
    enc1 = (jnp.dot(sincos, w1sc_ref[...], preferred_element_type=jnp.float32)
            + dist * w1dist_ref[...]
            + jnp.dot(direction, w1dir_ref[...], preferred_element_type=jnp.float32)
            + pb1_ref[...])
    pos = jnp.dot(_silu(enc1), wp2_ref[...], preferred_element_type=jnp.float32) + pb2_ref[...]
    ni = _silu(ni_ref[...])
    m1 = _silu(jnp.dot(ni, ma_ref[...], preferred_element_type=jnp.float32)
               + jnp.dot(pos, mb_ref[...], preferred_element_type=jnp.float32)
               + jnp.dot(ea_ref[...], mc_ref[...], preferred_element_type=jnp.float32)
               + mb1_ref[...])
    m = _silu(jnp.dot(m1, mw2_ref[...], preferred_element_type=jnp.float32) + mb2_ref[...])
    m_ref[...] = m
    s1 = _silu(jnp.dot(m, cw1_ref[...], preferred_element_type=jnp.float32) + cb1_ref[...])
    scale16 = jnp.dot(s1, c2r_ref[...], preferred_element_type=jnp.float32)
    lane = lax.broadcasted_iota(jnp.int32, (_P2C, 16), 1)
    ones12 = jnp.where(lane == 12, 1.0, 0.0).astype(jnp.float32)
    t_ref[...] = cd * scale16 + ones12


def _p2(pre_ni, cd16, edge_attr, w1s, b1, w2p, b2p, scm, ph, w1sc, w1dist,
        w1dir, pb1, wp2, pb2, ma, mb, mc, mb1, mw2, mb2, cw1, cb1, c2r):
    grid = E // _P2C
    full = lambda shape: pl.BlockSpec(shape, lambda i: tuple(0 for _ in shape))
    return pl.pallas_call(
        _p2_body,
        grid=(grid,),
        in_specs=[
            pl.BlockSpec((_P2C, D), lambda i: (i, 0)),
            pl.BlockSpec((_P2C, 16), lambda i: (i, 0)),
            pl.BlockSpec((_P2C, ED), lambda i: (i, 0)),
            full((16, 48)), full((16, 48)), full((48, 16)),
            full((16, D)), full((1, D)), full((D, D)), full((1, D)),
            full((D, 192)), full((1, 192)), full((192, 32)), full((1, 32)),
            full((D, 32)), full((1, 32)),
            full((32, 32)), full((1, 32)),
            full((D, D)), full((32, D)), full((ED, D)), full((1, D)),
            full((D, D)), full((1, D)),
            full((D, D)), full((1, D)), full((D, 16)),
        ],
        out_specs=(
            pl.BlockSpec((_P2C, D), lambda i: (i, 0)),
            pl.BlockSpec((_P2C, 16), lambda i: (i, 0)),
        ),
        out_shape=(
            jax.ShapeDtypeStruct((E, D), jnp.float32),
            jax.ShapeDtypeStruct((E, 16), jnp.float32),
        ),
    )(pre_ni, cd16, edge_attr, jnp.asarray(_G1), jnp.asarray(_G2),
      jnp.asarray(_S), w1s, b1, w2p, b2p, scm, ph, w1sc, w1dist,
      w1dir, pb1, wp2, pb2, ma, mb, mc, mb1, mw2, mb2, cw1, cb1, c2r)


def _scatter_body(row_hbm, m_hbm, t_hbm, pm_out, pt_out,
                  idx, idx8, bm, bt, shm, sht):
    cid = lax.axis_index("c")
    sid = lax.axis_index("s")
    wid = sid * NSC + cid
    base = wid * EW
    roff = sid * ROWS_PER_SUB

    zero16 = jnp.zeros((16,), jnp.float32)

    def zb(rr, carry):
        for j in range(8):
            bm[rr, pl.ds(16 * j, 16)] = zero16
        bt[rr, :] = zero16
        return carry

    lax.fori_loop(0, CH, zb, 0)
    for k in range(ROWS_PER_SUB // CH):
        sl = pl.ds(roff + k * CH, CH)
        pltpu.sync_copy(bm, shm.at[sl])
        pltpu.sync_copy(bt, sht.at[sl])
    plsc.subcore_barrier()

    def loop_body(k, carry):
        goff = base + k * CH
        pltpu.sync_copy(row_hbm.at[pl.ds(goff, CH)], idx)
        pltpu.sync_copy(m_hbm.at[pl.ds(goff, CH)], bm)
        pltpu.sync_copy(t_hbm.at[pl.ds(goff, CH)], bt)
        pltpu.sync_copy(bm, shm.at[idx], add=True)
        pltpu.sync_copy(bt, sht.at[idx], add=True)
        return carry

    lax.fori_loop(0, NFULL, loop_body, 0)
    goff = base + NFULL * CH
    pltpu.sync_copy(row_hbm.at[pl.ds(goff, TAIL)], idx8)
    pltpu.sync_copy(m_hbm.at[pl.ds(goff, TAIL)], bm.at[pl.ds(0, TAIL)])
    pltpu.sync_copy(t_hbm.at[pl.ds(goff, TAIL)], bt.at[pl.ds(0, TAIL)])
    pltpu.sync_copy(bm.at[pl.ds(0, TAIL)], shm.at[idx8], add=True)
    pltpu.sync_copy(bt.at[pl.ds(0, TAIL)], sht.at[idx8], add=True)
    plsc.subcore_barrier()

    for k in range(ROWS_PER_SUB // CH):
        sl = pl.ds(roff + k * CH, CH)
        pltpu.sync_copy(shm.at[sl], bm)
        pltpu.sync_copy(sht.at[sl], bt)
        pltpu.sync_copy(bm, pm_out.at[cid, sl])
        pltpu.sync_copy(bt, pt_out.at[cid, sl])


def _scatter_sc(row, m, t16):
    mesh = plsc.VectorSubcoreMesh(
        core_axis_name="c", subcore_axis_name="s",
        num_cores=NSC, num_subcores=NSUB)
    fn = functools.partial(
        pl.kernel,
        out_type=(
            jax.ShapeDtypeStruct((NSC, NP, D), jnp.float32),
            jax.ShapeDtypeStruct((NSC, NP, 16), jnp.float32),
        ),
        mesh=mesh,
        scratch_types=[
            pltpu.VMEM((CH,), jnp.int32),
            pltpu.VMEM((TAIL,), jnp.int32),
            pltpu.VMEM((CH, D), jnp.float32),
            pltpu.VMEM((CH, 16), jnp.float32),
            pltpu.VMEM_SHARED((NP, D), jnp.float32),
            pltpu.VMEM_SHARED((NP, 16), jnp.float32),
        ],
        compiler_params=pltpu.CompilerParams(use_tc_tiling_on_sc=False),
    )(_scatter_body)
    return fn(row, m, t16)


def _k5_body(x_ref, c16_ref, pm_ref, pt_ref, geo_ref,
             nx_ref, na_ref, nb1_ref, nw2_ref, nb2_ref, h_ref, co_ref):
    aggm = (pm_ref[0] + pm_ref[1])[:N]
    agg = jnp.dot(aggm, geo_ref[...], preferred_element_type=jnp.float32)
    x = x_ref[...]
    h1 = _silu(jnp.dot(x, nx_ref[...], preferred_element_type=jnp.float32)
               + jnp.dot(agg, na_ref[...], preferred_element_type=jnp.float32)
               + nb1_ref[...])
    h_ref[...] = x + jnp.dot(h1, nw2_ref[...], preferred_element_type=jnp.float32) + nb2_ref[...]
    qt = (pt_ref[0] + pt_ref[1])[:N]
    cnt = qt[:, 12:13]
    lane = lax.broadcasted_iota(jnp.int32, (N, 16), 1)
    tr = jnp.where(lane < 12, qt, 0.0)
    co_ref[...] = c16_ref[...] + tr * (1.0 / jnp.maximum(cnt, 1.0))


def _k5(x, coord16, pm, pt, geo, nx, na, nb1, nw2, nb2):
    return pl.pallas_call(
        _k5_body,
        out_shape=(
            jax.ShapeDtypeStruct((N, D), jnp.float32),
            jax.ShapeDtypeStruct((N, 16), jnp.float32),
        ),
    )(x, coord16, pm, pt, geo, nx, na, nb1, nw2, nb2)


def kernel(x, coord, edge_attr, edge_index, pe_w1, pe_b1, pe_w2, pe_b2,
           pe_p_w1, pe_p_b1, pe_p_w2, pe_p_b2, ni_w, ni_b, mm_w1, mm_b1,
           mm_w2, mm_b2, geo_w, nm_w1, nm_b1, nm_w2, nm_b2, cm_w1, cm_b1,
           cm_w2, frequencies):
    f32 = jnp.float32
    row = edge_index[0]
    col = edge_index[1]
    coord16 = jnp.pad(coord.reshape(N, 12), ((0, 0), (0, 4))).astype(f32)

    ni_wT = ni_w.T
    ws, wt = ni_wT[:D], ni_wT[D:]
    nib = ni_b.reshape(1, D)

    pe_w1T = pe_w1.T
    b1 = pe_b1.reshape(1, D)
    w2p = jnp.pad(pe_w2.T, ((0, 0), (0, D - 3)))
    b2p = jnp.pad(pe_b2.reshape(1, 3), ((0, 0), (0, D - 3)))

    m3 = jnp.kron(jnp.eye(3, dtype=f32), frequencies.reshape(1, NFB))
    scm = jnp.pad(jnp.concatenate([m3, m3], axis=1), ((0, D - 3), (0, 0)))
    ph = jnp.concatenate([jnp.zeros((1, 96), f32),
                          jnp.full((1, 96), np.float32(np.pi / 2))], axis=1)

    w1T = pe_p_w1.T
    w1sc = w1T[:192]
    w1dist = w1T[192:193]
    w1dir = jnp.pad(w1T[193:196], ((0, D - 3), (0, 0)))
    pb1 = pe_p_b1.reshape(1, 32)
    wp2 = pe_p_w2.T
    pb2 = pe_p_b2.reshape(1, 32)

    mm_w1T = mm_w1.T
    ma, mb, mc = mm_w1T[:D], mm_w1T[D:D + 32], mm_w1T[D + 32:]
    mb1 = mm_b1.reshape(1, D)
    mw2 = mm_w2.T
    mb2 = mm_b2.reshape(1, D)

    cw1 = cm_w1.T
    cb1 = cm_b1.reshape(1, D)
    c2r = jnp.pad(cm_w2.T @ jnp.asarray(_R), ((0, 0), (0, 4)))

    geo = geo_w.T
    nm_w1T = nm_w1.T
    nx, na = nm_w1T[:D], nm_w1T[D:]
    nb1 = nm_b1.reshape(1, D)
    nw2 = nm_w2.T
    nb2 = nm_b2.reshape(1, D)

    xs, xt = _k1(x, ws, wt, nib)
    pre_ni, cd16 = _gather_sc(row, col, xs, xt, coord16)
    sumsq = _p1(cd16)
    nrm = jnp.sqrt(sumsq.reshape(16))
    w1s = pe_w1T * (1.0 / jnp.maximum(nrm, 1e-12))[:, None]
    m, t16 = _p2(pre_ni, cd16, edge_attr, w1s, b1, w2p, b2p, scm, ph,
                 w1sc, w1dist, w1dir, pb1, wp2, pb2,
                 ma, mb, mc, mb1, mw2, mb2, cw1, cb1, c2r)
    pm, pt = _scatter_sc(row, m, t16)
    h_out, co16 = _k5(x, coord16, pm, pt, geo, nx, na, nb1, nw2, nb2)
    coord_out = co16[:, :12].reshape(N, NC, 3)
    return (h_out, coord_out)


# speedup vs baseline: 32.5172x; 1.0009x over previous
import functools

import numpy as np
import jax
import jax.numpy as jnp
from jax import lax
from jax.experimental import pallas as pl
from jax.experimental.pallas import tpu as pltpu
from jax.experimental.pallas import tpu_sc as plsc

N = 10000
E = 160000
D = 128
H = 128
NC = 4
ED = 16
NFB = 32

NP = 10240
NSC = 2
NSUB = 16
NW = NSC * NSUB
EW = E // NW
CH = 128
NFULL = EW // CH
TAIL = EW - NFULL * CH

ROWS_PER_SUB = NP // NSUB


def _silu(v):
    return v * jax.nn.sigmoid(v)


_TWO_PI = np.float32(2.0 * np.pi)
_INV_TWO_PI = np.float32(1.0 / (2.0 * np.pi))
_PI = np.float32(np.pi)
_HALF_PI = np.float32(np.pi / 2.0)


def _fast_sin(x):
    y = x * _INV_TWO_PI
    t = (y - jnp.round(y)) * _TWO_PI
    at = jnp.abs(t)
    f = jnp.where(at > _HALF_PI, _PI - at, at)
    f2 = f * f
    p = f * (1.0 + f2 * (np.float32(-1.0 / 6.0)
                         + f2 * (np.float32(1.0 / 120.0)
                                 + f2 * (np.float32(-1.0 / 5040.0)
                                         + f2 * np.float32(1.0 / 362880.0)))))
    return jnp.where(t < 0.0, -p, p)


def _build_consts():
    g1 = np.zeros((16, 48), np.float32)
    g2 = np.zeros((16, 48), np.float32)
    s = np.zeros((48, 16), np.float32)
    for i in range(4):
        for k in range(4):
            for j in range(3):
                p = (i * 4 + k) * 3 + j
                g1[3 * i + j, p] = 1.0
                g2[3 * k + j, p] = 1.0
                s[p, i * 4 + k] = 1.0
    r = np.zeros((4, 12), np.float32)
    for i in range(4):
        for j in range(3):
            r[i, 3 * i + j] = 1.0
    return g1, g2, s, r


_G1, _G2, _S, _R = _build_consts()


def _k1_body(x_ref, ws_ref, wt_ref, b_ref, xs_ref, xt_ref):
    x = x_ref[...]
    xs_ref[...] = jnp.dot(x, ws_ref[...], preferred_element_type=jnp.float32) + b_ref[...]
    xt_ref[...] = jnp.dot(x, wt_ref[...], preferred_element_type=jnp.float32)


def _k1(x, ws, wt, nib):
    return pl.pallas_call(
        _k1_body,
        out_shape=(
            jax.ShapeDtypeStruct((N, D), jnp.float32),
            jax.ShapeDtypeStruct((N, D), jnp.float32),
        ),
    )(x, ws, wt, nib)


def _gather_body(row_hbm, col_hbm, xs_hbm, xt_hbm, cp_hbm, ni_out, cd_out,
                 ridx0, cidx0, ridx1, cidx1, r8, c8,
                 a0, b0, p0, q0, a1, b1, p1, q1, sg0, sg1, so0, so1):
    wid = lax.axis_index("s") * NSC + lax.axis_index("c")
    base = wid * EW

    set0 = (ridx0, cidx0, a0, b0, p0, q0, sg0, so0)
    set1 = (ridx1, cidx1, a1, b1, p1, q1, sg1, so1)

    def load_idx(goff, st):
        ridx, cidx = st[0], st[1]
        pltpu.sync_copy(row_hbm.at[pl.ds(goff, CH)], ridx)
        pltpu.sync_copy(col_hbm.at[pl.ds(goff, CH)], cidx)

    def issue(st):
        ridx, cidx, a, b, p, q, sg, _ = st
        pltpu.async_copy(xs_hbm.at[ridx], a, sg)
        pltpu.async_copy(xt_hbm.at[cidx], b, sg)
        pltpu.async_copy(cp_hbm.at[ridx], p, sg)
        pltpu.async_copy(cp_hbm.at[cidx], q, sg)

    def wait_g(st):
        _, _, a, b, p, q, sg, _ = st
        pltpu.make_async_copy(xs_hbm.at[pl.ds(0, CH)], a, sg).wait()
        pltpu.make_async_copy(xs_hbm.at[pl.ds(0, CH)], b, sg).wait()
        pltpu.make_async_copy(cp_hbm.at[pl.ds(0, CH)], p, sg).wait()
        pltpu.make_async_copy(cp_hbm.at[pl.ds(0, CH)], q, sg).wait()

    def compute(st, size):
        _, _, a, b, p, q, _, _ = st

        def body(rr, carry):
            for j in range(8):
                sl = pl.ds(16 * j, 16)
                a[rr, sl] = a[rr, sl] + b[rr, sl]
            p[rr, :] = p[rr, :] - q[rr, :]
            return carry

        lax.fori_loop(0, size, body, 0)

    def out_async(goff, st):
        _, _, a, _, p, _, _, so = st
        pltpu.async_copy(a, ni_out.at[pl.ds(goff, CH)], so)
        pltpu.async_copy(p, cd_out.at[pl.ds(goff, CH)], so)

    def wait_o(st):
        _, _, a, _, p, _, _, so = st
        pltpu.make_async_copy(a, ni_out.at[pl.ds(0, CH)], so).wait()
        pltpu.make_async_copy(p, cd_out.at[pl.ds(0, CH)], so).wait()

    load_idx(base, set0)
    issue(set0)

    def loop_body(h, carry):
        c0 = base + (2 * h) * CH
        c1 = base + (2 * h + 1) * CH
        wait_g(set0)
        load_idx(c1, set1)
        issue(set1)
        compute(set0, CH)
        out_async(c0, set0)
        wait_g(set1)
        load_idx(c0 + 2 * CH, set0)
        wait_o(set0)
        issue(set0)
        compute(set1, CH)
        out_async(c1, set1)
        wait_o(set1)
        return carry

    lax.fori_loop(0, (NFULL - 1) // 2, loop_body, 0)

    gl = base + (NFULL - 1) * CH
    wait_g(set0)
    compute(set0, CH)
    out_async(gl, set0)
    wait_o(set0)

    gt = base + NFULL * CH
    pltpu.sync_copy(row_hbm.at[pl.ds(gt, TAIL)], r8)
    pltpu.sync_copy(col_hbm.at[pl.ds(gt, TAIL)], c8)
    pltpu.async_copy(xs_hbm.at[r8], a0.at[pl.ds(0, TAIL)], sg0).wait()
    pltpu.async_copy(xt_hbm.at[c8], b0.at[pl.ds(0, TAIL)], sg0).wait()
    pltpu.async_copy(cp_hbm.at[r8], p0.at[pl.ds(0, TAIL)], sg0).wait()
    pltpu.async_copy(cp_hbm.at[c8], q0.at[pl.ds(0, TAIL)], sg0).wait()
    compute(set0, TAIL)
    pltpu.sync_copy(a0.at[pl.ds(0, TAIL)], ni_out.at[pl.ds(gt, TAIL)])
    pltpu.sync_copy(p0.at[pl.ds(0, TAIL)], cd_out.at[pl.ds(gt, TAIL)])


def _gather_sc(row, col, xs, xt, coord16):
    mesh = plsc.VectorSubcoreMesh(
        core_axis_name="c", subcore_axis_name="s",
        num_cores=NSC, num_subcores=NSUB)
    fn = functools.partial(
        pl.kernel,
        out_type=(
            jax.ShapeDtypeStruct((E, D), jnp.float32),
            jax.ShapeDtypeStruct((E, 16), jnp.float32),
        ),
        mesh=mesh,
        scratch_types=[
            pltpu.VMEM((CH,), jnp.int32),
            pltpu.VMEM((CH,), jnp.int32),
            pltpu.VMEM((CH,), jnp.int32),
            pltpu.VMEM((CH,), jnp.int32),
            pltpu.VMEM((TAIL,), jnp.int32),
            pltpu.VMEM((TAIL,), jnp.int32),
            pltpu.VMEM((CH, D), jnp.float32),
            pltpu.VMEM((CH, D), jnp.float32),
            pltpu.VMEM((CH, 16), jnp.float32),
            pltpu.VMEM((CH, 16), jnp.float32),
            pltpu.VMEM((CH, D), jnp.float32),
            pltpu.VMEM((CH, D), jnp.float32),
            pltpu.VMEM((CH, 16), jnp.float32),
            pltpu.VMEM((CH, 16), jnp.float32),
            pltpu.SemaphoreType.DMA,
            pltpu.SemaphoreType.DMA,
            pltpu.SemaphoreType.DMA,
            pltpu.SemaphoreType.DMA,
        ],
        compiler_params=pltpu.CompilerParams(use_tc_tiling_on_sc=False),
    )(_gather_body)
    return fn(row, col, xs, xt, coord16)


_P1C = 2000


def _p1_body(cd_ref, g1_ref, g2_ref, s_ref, out_ref):
    cd = cd_ref[...]
    u = jnp.dot(cd, g1_ref[...], preferred_element_type=jnp.float32)
    v = jnp.dot(cd, g2_ref[...], preferred_element_type=jnp.float32)
    rad = jnp.dot(u * v, s_ref[...], preferred_element_type=jnp.float32)
    part = jnp.sum(rad * rad, axis=0, keepdims=True)

    @pl.when(pl.program_id(0) == 0)
    def _():
        out_ref[...] = jnp.zeros_like(out_ref)

    out_ref[...] += part


def _p1(cd16):
    grid = E // _P1C
    return pl.pallas_call(
        _p1_body,
        grid=(grid,),
        in_specs=[
            pl.BlockSpec((_P1C, 16), lambda i: (i, 0)),
            pl.BlockSpec((16, 48), lambda i: (0, 0)),
            pl.BlockSpec((16, 48), lambda i: (0, 0)),
            pl.BlockSpec((48, 16), lambda i: (0, 0)),
        ],
        out_specs=pl.BlockSpec((1, 16), lambda i: (0, 0)),
        out_shape=jax.ShapeDtypeStruct((1, 16), jnp.float32),
    )(cd16, jnp.asarray(_G1), jnp.asarray(_G2), jnp.asarray(_S))


_P2C = 1000


def _p2_body(ni_ref, cd_ref, ea_ref, g1_ref, g2_ref, s_ref,
             w1s_ref, b1_ref, w2_ref, b2_ref,
             scm_ref, ph_ref, w1sc_ref, w1dist_ref, w1dir_ref, pb1_ref,
             wp2_ref, pb2_ref,
             ma_ref, mb_ref, mc_ref, mb1_ref, mw2_ref, mb2_ref,
             cw1_ref, cb1_ref, c2r_ref,
             m_ref, t_ref):
    cd = cd_ref[...]
    u = jnp.dot(cd, g1_ref[...], preferred_element_type=jnp.float32)
    v = jnp.dot(cd, g2_ref[...], preferred_element_type=jnp.float32)
    rad = jnp.dot(u * v, s_ref[...], preferred_element_type=jnp.float32)
    h1 = _silu(jnp.dot(rad, w1s_ref[...], preferred_element_type=jnp.float32) + b1_ref[...])
    cdiff = jnp.dot(h1, w2_ref[...], preferred_element_type=jnp.float32) + b2_ref[...]
    d2 = jnp.sum(cdiff * cdiff, axis=1, keepdims=True)
    dist = jnp.sqrt(d2)
    direction = cdiff * (1.0 / (dist + 1e-8))
    sincos = _fast_sin(jnp.dot(cdiff, scm_ref[...], preferred_element_type=jnp.float32) + ph_ref[...])
    enc1 = (jnp.dot(sincos, w1sc_ref[...], preferred_element_type=jnp.float32)
            + dist * w1dist_ref[...]
            + jnp.dot(direction, w1dir_ref[...], preferred_element_type=jnp.float32)
            + pb1_ref[...])
    pos = jnp.dot(_silu(enc1), wp2_ref[...], preferred_element_type=jnp.float32) + pb2_ref[...]
    ni = _silu(ni_ref[...])
    m1 = _silu(jnp.dot(ni, ma_ref[...], preferred_element_type=jnp.float32)
               + jnp.dot(pos, mb_ref[...], preferred_element_type=jnp.float32)
               + jnp.dot(ea_ref[...], mc_ref[...], preferred_element_type=jnp.float32)
               + mb1_ref[...])
    m = _silu(jnp.dot(m1, mw2_ref[...], preferred_element_type=jnp.float32) + mb2_ref[...])
    m_ref[...] = m
    s1 = _silu(jnp.dot(m, cw1_ref[...], preferred_element_type=jnp.float32) + cb1_ref[...])
    scale16 = jnp.dot(s1, c2r_ref[...], preferred_element_type=jnp.float32)
    lane = lax.broadcasted_iota(jnp.int32, (_P2C, 16), 1)
    ones12 = jnp.where(lane == 12, 1.0, 0.0).astype(jnp.float32)
    t_ref[...] = cd * scale16 + ones12


def _p2(pre_ni, cd16, edge_attr, w1s, b1, w2p, b2p, scm, ph, w1sc, w1dist,
        w1dir, pb1, wp2, pb2, ma, mb, mc, mb1, mw2, mb2, cw1, cb1, c2r):
    grid = E // _P2C
    full = lambda shape: pl.BlockSpec(shape, lambda i: tuple(0 for _ in shape))
    return pl.pallas_call(
        _p2_body,
        grid=(grid,),
        in_specs=[
            pl.BlockSpec((_P2C, D), lambda i: (i, 0)),
            pl.BlockSpec((_P2C, 16), lambda i: (i, 0)),
            pl.BlockSpec((_P2C, ED), lambda i: (i, 0)),
            full((16, 48)), full((16, 48)), full((48, 16)),
            full((16, D)), full((1, D)), full((D, D)), full((1, D)),
            full((D, 192)), full((1, 192)), full((192, 32)), full((1, 32)),
            full((D, 32)), full((1, 32)),
            full((32, 32)), full((1, 32)),
            full((D, D)), full((32, D)), full((ED, D)), full((1, D)),
            full((D, D)), full((1, D)),
            full((D, D)), full((1, D)), full((D, 16)),
        ],
        out_specs=(
            pl.BlockSpec((_P2C, D), lambda i: (i, 0)),
            pl.BlockSpec((_P2C, 16), lambda i: (i, 0)),
        ),
        out_shape=(
            jax.ShapeDtypeStruct((E, D), jnp.float32),
            jax.ShapeDtypeStruct((E, 16), jnp.float32),
        ),
    )(pre_ni, cd16, edge_attr, jnp.asarray(_G1), jnp.asarray(_G2),
      jnp.asarray(_S), w1s, b1, w2p, b2p, scm, ph, w1sc, w1dist,
      w1dir, pb1, wp2, pb2, ma, mb, mc, mb1, mw2, mb2, cw1, cb1, c2r)


def _scatter_body(row_hbm, m_hbm, t_hbm, pm_out, pt_out,
                  idx, idx8, bm, bt, shm, sht):
    cid = lax.axis_index("c")
    sid = lax.axis_index("s")
    wid = sid * NSC + cid
    base = wid * EW
    roff = sid * ROWS_PER_SUB

    zero16 = jnp.zeros((16,), jnp.float32)

    def zb(rr, carry):
        for j in range(8):
            bm[rr, pl.ds(16 * j, 16)] = zero16
        bt[rr, :] = zero16
        return carry

    lax.fori_loop(0, CH, zb, 0)
    for k in range(ROWS_PER_SUB // CH):
        sl = pl.ds(roff + k * CH, CH)
        pltpu.sync_copy(bm, shm.at[sl])
        pltpu.sync_copy(bt, sht.at[sl])
    plsc.subcore_barrier()

    def loop_body(k, carry):
        goff = base + k * CH
        pltpu.sync_copy(row_hbm.at[pl.ds(goff, CH)], idx)
        pltpu.sync_copy(m_hbm.at[pl.ds(goff, CH)], bm)
        pltpu.sync_copy(t_hbm.at[pl.ds(goff, CH)], bt)
        pltpu.sync_copy(bm, shm.at[idx], add=True)
        pltpu.sync_copy(bt, sht.at[idx], add=True)
        return carry

    lax.fori_loop(0, NFULL, loop_body, 0)
    goff = base + NFULL * CH
    pltpu.sync_copy(row_hbm.at[pl.ds(goff, TAIL)], idx8)
    pltpu.sync_copy(m_hbm.at[pl.ds(goff, TAIL)], bm.at[pl.ds(0, TAIL)])
    pltpu.sync_copy(t_hbm.at[pl.ds(goff, TAIL)], bt.at[pl.ds(0, TAIL)])
    pltpu.sync_copy(bm.at[pl.ds(0, TAIL)], shm.at[idx8], add=True)
    pltpu.sync_copy(bt.at[pl.ds(0, TAIL)], sht.at[idx8], add=True)
    plsc.subcore_barrier()

    for k in range(ROWS_PER_SUB // CH):
        sl = pl.ds(roff + k * CH, CH)
        pltpu.sync_copy(shm.at[sl], bm)
        pltpu.sync_copy(sht.at[sl], bt)
        pltpu.sync_copy(bm, pm_out.at[cid, sl])
        pltpu.sync_copy(bt, pt_out.at[cid, sl])


def _scatter_sc(row, m, t16):
    mesh = plsc.VectorSubcoreMesh(
        core_axis_name="c", subcore_axis_name="s",
        num_cores=NSC, num_subcores=NSUB)
    fn = functools.partial(
        pl.kernel,
        out_type=(
            jax.ShapeDtypeStruct((NSC, NP, D), jnp.float32),
            jax.ShapeDtypeStruct((NSC, NP, 16), jnp.float32),
        ),
        mesh=mesh,
        scratch_types=[
            pltpu.VMEM((CH,), jnp.int32),
            pltpu.VMEM((TAIL,), jnp.int32),
            pltpu.VMEM((CH, D), jnp.float32),
            pltpu.VMEM((CH, 16), jnp.float32),
            pltpu.VMEM_SHARED((NP, D), jnp.float32),
            pltpu.VMEM_SHARED((NP, 16), jnp.float32),
        ],
        compiler_params=pltpu.CompilerParams(use_tc_tiling_on_sc=False),
    )(_scatter_body)
    return fn(row, m, t16)


def _k5_body(x_ref, c16_ref, pm_ref, pt_ref, geo_ref,
             nx_ref, na_ref, nb1_ref, nw2_ref, nb2_ref, h_ref, co_ref):
    aggm = (pm_ref[0] + pm_ref[1])[:N]
    agg = jnp.dot(aggm, geo_ref[...], preferred_element_type=jnp.float32)
    x = x_ref[...]
    h1 = _silu(jnp.dot(x, nx_ref[...], preferred_element_type=jnp.float32)
               + jnp.dot(agg, na_ref[...], preferred_element_type=jnp.float32)
               + nb1_ref[...])
    h_ref[...] = x + jnp.dot(h1, nw2_ref[...], preferred_element_type=jnp.float32) + nb2_ref[...]
    qt = (pt_ref[0] + pt_ref[1])[:N]
    cnt = qt[:, 12:13]
    lane = lax.broadcasted_iota(jnp.int32, (N, 16), 1)
    tr = jnp.where(lane < 12, qt, 0.0)
    co_ref[...] = c16_ref[...] + tr * (1.0 / jnp.maximum(cnt, 1.0))


def _k5(x, coord16, pm, pt, geo, nx, na, nb1, nw2, nb2):
    return pl.pallas_call(
        _k5_body,
        out_shape=(
            jax.ShapeDtypeStruct((N, D), jnp.float32),
            jax.ShapeDtypeStruct((N, 16), jnp.float32),
        ),
    )(x, coord16, pm, pt, geo, nx, na, nb1, nw2, nb2)


def kernel(x, coord, edge_attr, edge_index, pe_w1, pe_b1, pe_w2, pe_b2,
           pe_p_w1, pe_p_b1, pe_p_w2, pe_p_b2, ni_w, ni_b, mm_w1, mm_b1,
           mm_w2, mm_b2, geo_w, nm_w1, nm_b1, nm_w2, nm_b2, cm_w1, cm_b1,
           cm_w2, frequencies):
    f32 = jnp.float32
    row = edge_index[0]
    col = edge_index[1]
    coord16 = jnp.pad(coord.reshape(N, 12), ((0, 0), (0, 4))).astype(f32)

    ni_wT = ni_w.T
    ws, wt = ni_wT[:D], ni_wT[D:]
    nib = ni_b.reshape(1, D)

    pe_w1T = pe_w1.T
    b1 = pe_b1.reshape(1, D)
    w2p = jnp.pad(pe_w2.T, ((0, 0), (0, D - 3)))
    b2p = jnp.pad(pe_b2.reshape(1, 3), ((0, 0), (0, D - 3)))

    m3 = jnp.kron(jnp.eye(3, dtype=f32), frequencies.reshape(1, NFB))
    scm = jnp.pad(jnp.concatenate([m3, m3], axis=1), ((0, D - 3), (0, 0)))
    ph = jnp.concatenate([jnp.zeros((1, 96), f32),
                          jnp.full((1, 96), np.float32(np.pi / 2))], axis=1)

    w1T = pe_p_w1.T
    w1sc = w1T[:192]
    w1dist = w1T[192:193]
    w1dir = jnp.pad(w1T[193:196], ((0, D - 3), (0, 0)))
    pb1 = pe_p_b1.reshape(1, 32)
    wp2 = pe_p_w2.T
    pb2 = pe_p_b2.reshape(1, 32)

    mm_w1T = mm_w1.T
    ma, mb, mc = mm_w1T[:D], mm_w1T[D:D + 32], mm_w1T[D + 32:]
    mb1 = mm_b1.reshape(1, D)
    mw2 = mm_w2.T
    mb2 = mm_b2.reshape(1, D)

    cw1 = cm_w1.T
    cb1 = cm_b1.reshape(1, D)
    c2r = jnp.pad(cm_w2.T @ jnp.asarray(_R), ((0, 0), (0, 4)))

    geo = geo_w.T
    nm_w1T = nm_w1.T
    nx, na = nm_w1T[:D], nm_w1T[D:]
    nb1 = nm_b1.reshape(1, D)
    nw2 = nm_w2.T
    nb2 = nm_b2.reshape(1, D)

    xs, xt = _k1(x, ws, wt, nib)
    pre_ni, cd16 = _gather_sc(row, col, xs, xt, coord16)
    sumsq = _p1(cd16)
    nrm = jnp.sqrt(sumsq.reshape(16))
    w1s = pe_w1T * (1.0 / jnp.maximum(nrm, 1e-12))[:, None]
    m, t16 = _p2(pre_ni, cd16, edge_attr, w1s, b1, w2p, b2p, scm, ph,
                 w1sc, w1dist, w1dir, pb1, wp2, pb2,
                 ma, mb, mc, mb1, mw2, mb2, cw1, cb1, c2r)
    pm, pt = _scatter_sc(row, m, t16)
    h_out, co16 = _k5(x, coord16, pm, pt, geo, nx, na, nb1, nw2, nb2)
    coord_out = co16[:, :12].reshape(N, NC, 3)
    return (h_out, coord_out)


# speedup vs baseline: 34.7523x; 1.0687x over previous
import functools

import numpy as np
import jax
import jax.numpy as jnp
from jax import lax
from jax.experimental import pallas as pl
from jax.experimental.pallas import tpu as pltpu
from jax.experimental.pallas import tpu_sc as plsc

N = 10000
E = 160000
D = 128
H = 128
NC = 4
ED = 16
NFB = 32

NP = 10240
NSC = 2
NSUB = 16
NW = NSC * NSUB
EW = E // NW
CH = 128
NFULL = EW // CH
TAIL = EW - NFULL * CH

ROWS_PER_SUB = NP // NSUB


def _silu(v):
    return v * jax.nn.sigmoid(v)


_TWO_PI = np.float32(2.0 * np.pi)
_INV_TWO_PI = np.float32(1.0 / (2.0 * np.pi))
_PI = np.float32(np.pi)
_HALF_PI = np.float32(np.pi / 2.0)


def _fast_sin(x):
    y = x * _INV_TWO_PI
    t = (y - jnp.round(y)) * _TWO_PI
    at = jnp.abs(t)
    f = jnp.where(at > _HALF_PI, _PI - at, at)
    f2 = f * f
    p = f * (1.0 + f2 * (np.float32(-1.0 / 6.0)
                         + f2 * (np.float32(1.0 / 120.0)
                                 + f2 * (np.float32(-1.0 / 5040.0)
                                         + f2 * np.float32(1.0 / 362880.0)))))
    return jnp.where(t < 0.0, -p, p)


def _build_consts():
    g1 = np.zeros((16, 48), np.float32)
    g2 = np.zeros((16, 48), np.float32)
    s = np.zeros((48, 16), np.float32)
    for i in range(4):
        for k in range(4):
            for j in range(3):
                p = (i * 4 + k) * 3 + j
                g1[3 * i + j, p] = 1.0
                g2[3 * k + j, p] = 1.0
                s[p, i * 4 + k] = 1.0
    r = np.zeros((4, 12), np.float32)
    for i in range(4):
        for j in range(3):
            r[i, 3 * i + j] = 1.0
    return g1, g2, s, r


_G1, _G2, _S, _R = _build_consts()


def _k1_body(x_ref, ws_ref, wt_ref, b_ref, xs_ref, xt_ref):
    x = x_ref[...]
    xs_ref[...] = jnp.dot(x, ws_ref[...], preferred_element_type=jnp.float32) + b_ref[...]
    xt_ref[...] = jnp.dot(x, wt_ref[...], preferred_element_type=jnp.float32)


def _k1(x, ws, wt, nib):
    return pl.pallas_call(
        _k1_body,
        out_shape=(
            jax.ShapeDtypeStruct((N, D), jnp.float32),
            jax.ShapeDtypeStruct((N, D), jnp.float32),
        ),
    )(x, ws, wt, nib)


def _gather_body(row_hbm, col_hbm, xs_hbm, xt_hbm, cp_hbm, ni_out, cd_out,
                 ridx0, cidx0, ridx1, cidx1, r8, c8,
                 a0, b0, p0, q0, a1, b1, p1, q1, sg0, sg1, so0, so1):
    wid = lax.axis_index("s") * NSC + lax.axis_index("c")
    base = wid * EW

    set0 = (ridx0, cidx0, a0, b0, p0, q0, sg0, so0)
    set1 = (ridx1, cidx1, a1, b1, p1, q1, sg1, so1)

    def load_idx(goff, st):
        ridx, cidx = st[0], st[1]
        pltpu.sync_copy(row_hbm.at[pl.ds(goff, CH)], ridx)
        pltpu.sync_copy(col_hbm.at[pl.ds(goff, CH)], cidx)

    def issue(st):
        ridx, cidx, a, b, p, q, sg, _ = st
        pltpu.async_copy(xs_hbm.at[ridx], a, sg)
        pltpu.async_copy(xt_hbm.at[cidx], b, sg)
        pltpu.async_copy(cp_hbm.at[ridx], p, sg)
        pltpu.async_copy(cp_hbm.at[cidx], q, sg)

    def wait_g(st):
        _, _, a, b, p, q, sg, _ = st
        pltpu.make_async_copy(xs_hbm.at[pl.ds(0, CH)], a, sg).wait()
        pltpu.make_async_copy(xs_hbm.at[pl.ds(0, CH)], b, sg).wait()
        pltpu.make_async_copy(cp_hbm.at[pl.ds(0, CH)], p, sg).wait()
        pltpu.make_async_copy(cp_hbm.at[pl.ds(0, CH)], q, sg).wait()

    def compute(st, size):
        _, _, a, b, p, q, _, _ = st

        def body(rr, carry):
            for j in range(8):
                sl = pl.ds(16 * j, 16)
                a[rr, sl] = a[rr, sl] + b[rr, sl]
            p[rr, :] = p[rr, :] - q[rr, :]
            return carry

        lax.fori_loop(0, size, body, 0)

    def out_async(goff, st):
        _, _, a, _, p, _, _, so = st
        pltpu.async_copy(a, ni_out.at[pl.ds(goff, CH)], so)
        pltpu.async_copy(p, cd_out.at[pl.ds(goff, CH)], so)

    def wait_o(st):
        _, _, a, _, p, _, _, so = st
        pltpu.make_async_copy(a, ni_out.at[pl.ds(0, CH)], so).wait()
        pltpu.make_async_copy(p, cd_out.at[pl.ds(0, CH)], so).wait()

    load_idx(base, set0)
    issue(set0)

    def loop_body(h, carry):
        c0 = base + (2 * h) * CH
        c1 = base + (2 * h + 1) * CH
        wait_g(set0)
        load_idx(c1, set1)
        issue(set1)
        compute(set0, CH)
        out_async(c0, set0)
        wait_g(set1)
        load_idx(c0 + 2 * CH, set0)
        wait_o(set0)
        issue(set0)
        compute(set1, CH)
        out_async(c1, set1)
        wait_o(set1)
        return carry

    lax.fori_loop(0, (NFULL - 1) // 2, loop_body, 0)

    gl = base + (NFULL - 1) * CH
    wait_g(set0)
    compute(set0, CH)
    out_async(gl, set0)
    wait_o(set0)

    gt = base + NFULL * CH
    pltpu.sync_copy(row_hbm.at[pl.ds(gt, TAIL)], r8)
    pltpu.sync_copy(col_hbm.at[pl.ds(gt, TAIL)], c8)
    pltpu.async_copy(xs_hbm.at[r8], a0.at[pl.ds(0, TAIL)], sg0).wait()
    pltpu.async_copy(xt_hbm.at[c8], b0.at[pl.ds(0, TAIL)], sg0).wait()
    pltpu.async_copy(cp_hbm.at[r8], p0.at[pl.ds(0, TAIL)], sg0).wait()
    pltpu.async_copy(cp_hbm.at[c8], q0.at[pl.ds(0, TAIL)], sg0).wait()
    compute(set0, TAIL)
    pltpu.sync_copy(a0.at[pl.ds(0, TAIL)], ni_out.at[pl.ds(gt, TAIL)])
    pltpu.sync_copy(p0.at[pl.ds(0, TAIL)], cd_out.at[pl.ds(gt, TAIL)])


def _gather_sc(row, col, xs, xt, coord16):
    mesh = plsc.VectorSubcoreMesh(
        core_axis_name="c", subcore_axis_name="s",
        num_cores=NSC, num_subcores=NSUB)
    fn = functools.partial(
        pl.kernel,
        out_type=(
            jax.ShapeDtypeStruct((E, D), jnp.float32),
            jax.ShapeDtypeStruct((E, 16), jnp.float32),
        ),
        mesh=mesh,
        scratch_types=[
            pltpu.VMEM((CH,), jnp.int32),
            pltpu.VMEM((CH,), jnp.int32),
            pltpu.VMEM((CH,), jnp.int32),
            pltpu.VMEM((CH,), jnp.int32),
            pltpu.VMEM((TAIL,), jnp.int32),
            pltpu.VMEM((TAIL,), jnp.int32),
            pltpu.VMEM((CH, D), jnp.float32),
            pltpu.VMEM((CH, D), jnp.float32),
            pltpu.VMEM((CH, 16), jnp.float32),
            pltpu.VMEM((CH, 16), jnp.float32),
            pltpu.VMEM((CH, D), jnp.float32),
            pltpu.VMEM((CH, D), jnp.float32),
            pltpu.VMEM((CH, 16), jnp.float32),
            pltpu.VMEM((CH, 16), jnp.float32),
            pltpu.SemaphoreType.DMA,
            pltpu.SemaphoreType.DMA,
            pltpu.SemaphoreType.DMA,
            pltpu.SemaphoreType.DMA,
        ],
        compiler_params=pltpu.CompilerParams(use_tc_tiling_on_sc=False),
    )(_gather_body)
    return fn(row, col, xs, xt, coord16)


_P1C = 2000


def _p1_body(cd_ref, g1_ref, g2_ref, s_ref, out_ref):
    cd = cd_ref[...]
    u = jnp.dot(cd, g1_ref[...], preferred_element_type=jnp.float32)
    v = jnp.dot(cd, g2_ref[...], preferred_element_type=jnp.float32)
    rad = jnp.dot(u * v, s_ref[...], preferred_element_type=jnp.float32)
    part = jnp.sum(rad * rad, axis=0, keepdims=True)

    @pl.when(pl.program_id(0) == 0)
    def _():
        out_ref[...] = jnp.zeros_like(out_ref)

    out_ref[...] += part


def _p1(cd16):
    grid = E // _P1C
    return pl.pallas_call(
        _p1_body,
        grid=(grid,),
        in_specs=[
            pl.BlockSpec((_P1C, 16), lambda i: (i, 0)),
            pl.BlockSpec((16, 48), lambda i: (0, 0)),
            pl.BlockSpec((16, 48), lambda i: (0, 0)),
            pl.BlockSpec((48, 16), lambda i: (0, 0)),
        ],
        out_specs=pl.BlockSpec((1, 16), lambda i: (0, 0)),
        out_shape=jax.ShapeDtypeStruct((1, 16), jnp.float32),
    )(cd16, jnp.asarray(_G1), jnp.asarray(_G2), jnp.asarray(_S))


_P2C = 1000


def _p2_body(ni_ref, cd_ref, ea_ref, g1_ref, g2_ref, s_ref,
             w1s_ref, b1_ref, w2_ref, b2_ref,
             scm_ref, ph_ref, w1sc_ref, w1dist_ref, w1dir_ref, pb1_ref,
             wp2_ref, pb2_ref,
             ma_ref, mb_ref, mc_ref, mb1_ref, mw2_ref, mb2_ref,
             cw1_ref, cb1_ref, c2r_ref,
             m_ref, t_ref):
    cd = cd_ref[...]
    u = jnp.dot(cd, g1_ref[...], preferred_element_type=jnp.float32)
    v = jnp.dot(cd, g2_ref[...], preferred_element_type=jnp.float32)
    rad = jnp.dot(u * v, s_ref[...], preferred_element_type=jnp.float32)
    h1 = _silu(jnp.dot(rad, w1s_ref[...], preferred_element_type=jnp.float32) + b1_ref[...])
    cdiff = jnp.dot(h1, w2_ref[...], preferred_element_type=jnp.float32) + b2_ref[...]
    d2 = jnp.sum(cdiff * cdiff, axis=1, keepdims=True)
    dist = jnp.sqrt(d2)
    direction = cdiff * (1.0 / (dist + 1e-8))
    sincos = _fast_sin(jnp.dot(cdiff, scm_ref[...], preferred_element_type=jnp.float32) + ph_ref[...])
    enc1 = (jnp.dot(sincos, w1sc_ref[...], preferred_element_type=jnp.float32)
            + dist * w1dist_ref[...]
            + jnp.dot(direction, w1dir_ref[...], preferred_element_type=jnp.float32)
            + pb1_ref[...])
    pos = jnp.dot(_silu(enc1), wp2_ref[...], preferred_element_type=jnp.float32) + pb2_ref[...]
    ni = _silu(ni_ref[...])
    m1 = _silu(jnp.dot(ni, ma_ref[...], preferred_element_type=jnp.float32)
               + jnp.dot(pos, mb_ref[...], preferred_element_type=jnp.float32)
               + jnp.dot(ea_ref[...], mc_ref[...], preferred_element_type=jnp.float32)
               + mb1_ref[...])
    m = _silu(jnp.dot(m1, mw2_ref[...], preferred_element_type=jnp.float32) + mb2_ref[...])
    m_ref[...] = m
    s1 = _silu(jnp.dot(m, cw1_ref[...], preferred_element_type=jnp.float32) + cb1_ref[...])
    scale16 = jnp.dot(s1, c2r_ref[...], preferred_element_type=jnp.float32)
    lane = lax.broadcasted_iota(jnp.int32, (_P2C, 16), 1)
    ones12 = jnp.where(lane == 12, 1.0, 0.0).astype(jnp.float32)
    t_ref[...] = cd * scale16 + ones12


def _p2(pre_ni, cd16, edge_attr, w1s, b1, w2p, b2p, scm, ph, w1sc, w1dist,
        w1dir, pb1, wp2, pb2, ma, mb, mc, mb1, mw2, mb2, cw1, cb1, c2r):
    grid = E // _P2C
    full = lambda shape: pl.BlockSpec(shape, lambda i: tuple(0 for _ in shape))
    return pl.pallas_call(
        _p2_body,
        grid=(grid,),
        in_specs=[
            pl.BlockSpec((_P2C, D), lambda i: (i, 0)),
            pl.BlockSpec((_P2C, 16), lambda i: (i, 0)),
            pl.BlockSpec((_P2C, ED), lambda i: (i, 0)),
            full((16, 48)), full((16, 48)), full((48, 16)),
            full((16, D)), full((1, D)), full((D, D)), full((1, D)),
            full((D, 192)), full((1, 192)), full((192, 32)), full((1, 32)),
            full((D, 32)), full((1, 32)),
            full((32, 32)), full((1, 32)),
            full((D, D)), full((32, D)), full((ED, D)), full((1, D)),
            full((D, D)), full((1, D)),
            full((D, D)), full((1, D)), full((D, 16)),
        ],
        out_specs=(
            pl.BlockSpec((_P2C, D), lambda i: (i, 0)),
            pl.BlockSpec((_P2C, 16), lambda i: (i, 0)),
        ),
        out_shape=(
            jax.ShapeDtypeStruct((E, D), jnp.float32),
            jax.ShapeDtypeStruct((E, 16), jnp.float32),
        ),
    )(pre_ni, cd16, edge_attr, jnp.asarray(_G1), jnp.asarray(_G2),
      jnp.asarray(_S), w1s, b1, w2p, b2p, scm, ph, w1sc, w1dist,
      w1dir, pb1, wp2, pb2, ma, mb, mc, mb1, mw2, mb2, cw1, cb1, c2r)


def _scatter_body(row_hbm, m_hbm, t_hbm, pm_out, pt_out,
                  idx0, idx8, bm0, bt0, idx1, bm1, bt1, sl0, sl1, shm, sht):
    cid = lax.axis_index("c")
    sid = lax.axis_index("s")
    wid = sid * NSC + cid
    base = wid * EW
    roff = sid * ROWS_PER_SUB

    set0 = (idx0, bm0, bt0, sl0)
    set1 = (idx1, bm1, bt1, sl1)

    zero16 = jnp.zeros((16,), jnp.float32)

    def zb(rr, carry):
        for j in range(8):
            bm0[rr, pl.ds(16 * j, 16)] = zero16
        bt0[rr, :] = zero16
        return carry

    lax.fori_loop(0, CH, zb, 0)
    for k in range(ROWS_PER_SUB // CH):
        sl = pl.ds(roff + k * CH, CH)
        pltpu.sync_copy(bm0, shm.at[sl])
        pltpu.sync_copy(bt0, sht.at[sl])
    plsc.subcore_barrier()

    def issue_load(goff, st):
        idx, bm, bt, sem = st
        pltpu.async_copy(row_hbm.at[pl.ds(goff, CH)], idx, sem)
        pltpu.async_copy(m_hbm.at[pl.ds(goff, CH)], bm, sem)
        pltpu.async_copy(t_hbm.at[pl.ds(goff, CH)], bt, sem)

    def wait_load(st):
        idx, bm, bt, sem = st
        pltpu.make_async_copy(row_hbm.at[pl.ds(0, CH)], idx, sem).wait()
        pltpu.make_async_copy(m_hbm.at[pl.ds(0, CH)], bm, sem).wait()
        pltpu.make_async_copy(t_hbm.at[pl.ds(0, CH)], bt, sem).wait()

    def scat(st):
        idx, bm, bt, _ = st
        pltpu.sync_copy(bm, shm.at[idx], add=True)
        pltpu.sync_copy(bt, sht.at[idx], add=True)

    issue_load(base, set0)

    def loop_body(h, carry):
        c0 = base + (2 * h) * CH
        c1 = base + (2 * h + 1) * CH
        wait_load(set0)
        issue_load(c1, set1)
        scat(set0)
        issue_load(c0 + 2 * CH, set0)
        wait_load(set1)
        scat(set1)
        return carry

    lax.fori_loop(0, (NFULL - 1) // 2, loop_body, 0)
    wait_load(set0)
    scat(set0)

    goff = base + NFULL * CH
    pltpu.sync_copy(row_hbm.at[pl.ds(goff, TAIL)], idx8)
    pltpu.sync_copy(m_hbm.at[pl.ds(goff, TAIL)], bm0.at[pl.ds(0, TAIL)])
    pltpu.sync_copy(t_hbm.at[pl.ds(goff, TAIL)], bt0.at[pl.ds(0, TAIL)])
    pltpu.sync_copy(bm0.at[pl.ds(0, TAIL)], shm.at[idx8], add=True)
    pltpu.sync_copy(bt0.at[pl.ds(0, TAIL)], sht.at[idx8], add=True)
    plsc.subcore_barrier()

    for k in range(ROWS_PER_SUB // CH):
        sl = pl.ds(roff + k * CH, CH)
        pltpu.sync_copy(shm.at[sl], bm0)
        pltpu.sync_copy(sht.at[sl], bt0)
        pltpu.sync_copy(bm0, pm_out.at[cid, sl])
        pltpu.sync_copy(bt0, pt_out.at[cid, sl])


def _scatter_sc(row, m, t16):
    mesh = plsc.VectorSubcoreMesh(
        core_axis_name="c", subcore_axis_name="s",
        num_cores=NSC, num_subcores=NSUB)
    fn = functools.partial(
        pl.kernel,
        out_type=(
            jax.ShapeDtypeStruct((NSC, NP, D), jnp.float32),
            jax.ShapeDtypeStruct((NSC, NP, 16), jnp.float32),
        ),
        mesh=mesh,
        scratch_types=[
            pltpu.VMEM((CH,), jnp.int32),
            pltpu.VMEM((TAIL,), jnp.int32),
            pltpu.VMEM((CH, D), jnp.float32),
            pltpu.VMEM((CH, 16), jnp.float32),
            pltpu.VMEM((CH,), jnp.int32),
            pltpu.VMEM((CH, D), jnp.float32),
            pltpu.VMEM((CH, 16), jnp.float32),
            pltpu.SemaphoreType.DMA,
            pltpu.SemaphoreType.DMA,
            pltpu.VMEM_SHARED((NP, D), jnp.float32),
            pltpu.VMEM_SHARED((NP, 16), jnp.float32),
        ],
        compiler_params=pltpu.CompilerParams(use_tc_tiling_on_sc=False),
    )(_scatter_body)
    return fn(row, m, t16)


def _k5_body(x_ref, c16_ref, pm_ref, pt_ref, geo_ref,
             nx_ref, na_ref, nb1_ref, nw2_ref, nb2_ref, h_ref, co_ref):
    aggm = (pm_ref[0] + pm_ref[1])[:N]
    agg = jnp.dot(aggm, geo_ref[...], preferred_element_type=jnp.float32)
    x = x_ref[...]
    h1 = _silu(jnp.dot(x, nx_ref[...], preferred_element_type=jnp.float32)
               + jnp.dot(agg, na_ref[...], preferred_element_type=jnp.float32)
               + nb1_ref[...])
    h_ref[...] = x + jnp.dot(h1, nw2_ref[...], preferred_element_type=jnp.float32) + nb2_ref[...]
    qt = (pt_ref[0] + pt_ref[1])[:N]
    cnt = qt[:, 12:13]
    lane = lax.broadcasted_iota(jnp.int32, (N, 16), 1)
    tr = jnp.where(lane < 12, qt, 0.0)
    co_ref[...] = c16_ref[...] + tr * (1.0 / jnp.maximum(cnt, 1.0))


def _k5(x, coord16, pm, pt, geo, nx, na, nb1, nw2, nb2):
    return pl.pallas_call(
        _k5_body,
        out_shape=(
            jax.ShapeDtypeStruct((N, D), jnp.float32),
            jax.ShapeDtypeStruct((N, 16), jnp.float32),
        ),
    )(x, coord16, pm, pt, geo, nx, na, nb1, nw2, nb2)


def kernel(x, coord, edge_attr, edge_index, pe_w1, pe_b1, pe_w2, pe_b2,
           pe_p_w1, pe_p_b1, pe_p_w2, pe_p_b2, ni_w, ni_b, mm_w1, mm_b1,
           mm_w2, mm_b2, geo_w, nm_w1, nm_b1, nm_w2, nm_b2, cm_w1, cm_b1,
           cm_w2, frequencies):
    f32 = jnp.float32
    row = edge_index[0]
    col = edge_index[1]
    coord16 = jnp.pad(coord.reshape(N, 12), ((0, 0), (0, 4))).astype(f32)

    ni_wT = ni_w.T
    ws, wt = ni_wT[:D], ni_wT[D:]
    nib = ni_b.reshape(1, D)

    pe_w1T = pe_w1.T
    b1 = pe_b1.reshape(1, D)
    w2p = jnp.pad(pe_w2.T, ((0, 0), (0, D - 3)))
    b2p = jnp.pad(pe_b2.reshape(1, 3), ((0, 0), (0, D - 3)))

    m3 = jnp.kron(jnp.eye(3, dtype=f32), frequencies.reshape(1, NFB))
    scm = jnp.pad(jnp.concatenate([m3, m3], axis=1), ((0, D - 3), (0, 0)))
    ph = jnp.concatenate([jnp.zeros((1, 96), f32),
                          jnp.full((1, 96), np.float32(np.pi / 2))], axis=1)

    w1T = pe_p_w1.T
    w1sc = w1T[:192]
    w1dist = w1T[192:193]
    w1dir = jnp.pad(w1T[193:196], ((0, D - 3), (0, 0)))
    pb1 = pe_p_b1.reshape(1, 32)
    wp2 = pe_p_w2.T
    pb2 = pe_p_b2.reshape(1, 32)

    mm_w1T = mm_w1.T
    ma, mb, mc = mm_w1T[:D], mm_w1T[D:D + 32], mm_w1T[D + 32:]
    mb1 = mm_b1.reshape(1, D)
    mw2 = mm_w2.T
    mb2 = mm_b2.reshape(1, D)

    cw1 = cm_w1.T
    cb1 = cm_b1.reshape(1, D)
    c2r = jnp.pad(cm_w2.T @ jnp.asarray(_R), ((0, 0), (0, 4)))

    geo = geo_w.T
    nm_w1T = nm_w1.T
    nx, na = nm_w1T[:D], nm_w1T[D:]
    nb1 = nm_b1.reshape(1, D)
    nw2 = nm_w2.T
    nb2 = nm_b2.reshape(1, D)

    xs, xt = _k1(x, ws, wt, nib)
    pre_ni, cd16 = _gather_sc(row, col, xs, xt, coord16)
    sumsq = _p1(cd16)
    nrm = jnp.sqrt(sumsq.reshape(16))
    w1s = pe_w1T * (1.0 / jnp.maximum(nrm, 1e-12))[:, None]
    m, t16 = _p2(pre_ni, cd16, edge_attr, w1s, b1, w2p, b2p, scm, ph,
                 w1sc, w1dist, w1dir, pb1, wp2, pb2,
                 ma, mb, mc, mb1, mw2, mb2, cw1, cb1, c2r)
    pm, pt = _scatter_sc(row, m, t16)
    h_out, co16 = _k5(x, coord16, pm, pt, geo, nx, na, nb1, nw2, nb2)
    coord_out = co16[:, :12].reshape(N, NC, 3)
    return (h_out, coord_out)
